# baseline (speedup 1.0000x reference)
import functools

import jax
import jax.numpy as jnp
from jax import lax
from jax.experimental import pallas as pl
from jax.experimental.pallas import tpu as pltpu
from jax.experimental.pallas import tpu_sc as plsc

N = 10000
E = 320000
H = 128
HH = 64
NBLK = 1000
EBLK = 2000
B = 64
NUM_TILES = 16
NPAD = 10240
ROWS_PER_TILE = NPAD // NUM_TILES
ZROWS = 8
NUM_EBLK = E // B
F32 = jnp.float32


def _f32(x):
    return jax.ShapeDtypeStruct(x, F32)


def _fold_body(embW, embb, c1W, c1b, c2W, c2b, wf1, bf1, wf2, bf2):
    wf1[...] = jnp.dot(embW[...], c1W[...], preferred_element_type=F32)
    bf1[...] = jnp.dot(embb[...], c1W[...], preferred_element_type=F32) + c1b[...]
    wf2[...] = jnp.dot(embW[...], c2W[...], preferred_element_type=F32)
    bf2[...] = jnp.dot(embb[...], c2W[...], preferred_element_type=F32) + c2b[...]


def _fold_weights(embW, embb, c1W, c1b, c2W, c2b):
    return pl.pallas_call(
        _fold_body,
        out_shape=[_f32((16, H)), _f32((1, H)), _f32((16, H)), _f32((1, H))],
    )(embW, embb, c1W, c1b, c2W, c2b)


def _embed_body(h, w, b, out):
    out[...] = jnp.dot(h[...], w[...], preferred_element_type=F32) + b[...]


def _embed(h, w, b):
    return pl.pallas_call(
        _embed_body,
        grid=(N // NBLK,),
        in_specs=[
            pl.BlockSpec((NBLK, H), lambda i: (i, 0)),
            pl.BlockSpec((H, H), lambda i: (0, 0)),
            pl.BlockSpec((1, H), lambda i: (0, 0)),
        ],
        out_specs=pl.BlockSpec((NBLK, H), lambda i: (i, 0)),
        out_shape=_f32((N, H)),
    )(h, w, b)


def _node_mm_body(hh, aW, ab, bW, bb, dW, db_, eW, eb, ah, dbt, eht):
    x = hh[...]
    ah[...] = jnp.dot(x, aW[...], preferred_element_type=F32) + ab[...]
    Bh = jnp.dot(x, bW[...], preferred_element_type=F32) + bb[...]
    Dh = jnp.dot(x, dW[...], preferred_element_type=F32) + db_[...]
    Eh = jnp.dot(x, eW[...], preferred_element_type=F32) + eb[...]
    dbt[0] = jnp.concatenate([Dh[:, :HH], Bh[:, :HH]], axis=1)
    dbt[1] = jnp.concatenate([Dh[:, HH:], Bh[:, HH:]], axis=1)
    eht[...] = Eh


def _node_matmuls(hh, aW, ab, bW, bb, dW, db_, eW, eb):
    return pl.pallas_call(
        _node_mm_body,
        grid=(N // NBLK,),
        in_specs=[pl.BlockSpec((NBLK, H), lambda i: (i, 0))]
        + [pl.BlockSpec((H, H), lambda i: (0, 0)),
           pl.BlockSpec((1, H), lambda i: (0, 0))] * 4,
        out_specs=[
            pl.BlockSpec((NBLK, H), lambda i: (i, 0)),
            pl.BlockSpec((2, NBLK, H), lambda i: (0, i, 0)),
            pl.BlockSpec((NBLK, H), lambda i: (i, 0)),
        ],
        out_shape=[_f32((N, H)), _f32((2, N, H)), _f32((N, H))],
    )(hh, aW, ab, bW, bb, dW, db_, eW, eb)


def _edge_lin1_body(e, wf, bf, out):
    ce = jnp.dot(1.0 / e[...], wf[...], preferred_element_type=F32) + bf[...]
    out[0] = ce[:, :HH]
    out[1] = ce[:, HH:]


def _edge_linear1(e, wf, bf):
    return pl.pallas_call(
        _edge_lin1_body,
        grid=(E // EBLK,),
        in_specs=[
            pl.BlockSpec((EBLK, 16), lambda i: (i, 0)),
            pl.BlockSpec((16, H), lambda i: (0, 0)),
            pl.BlockSpec((1, H), lambda i: (0, 0)),
        ],
        out_specs=pl.BlockSpec((2, EBLK, HH), lambda i: (0, i, 0)),
        out_shape=_f32((2, E, HH)),
    )(e, wf, bf)


def _edge_lin2_body(e, enew, stats, wf, bf, cW, g, b, out):
    st = stats[:, 0, :]
    s1h0 = jnp.sum(st[:NUM_TILES, :HH], axis=0, keepdims=True)
    s2h0 = jnp.sum(st[:NUM_TILES, HH:], axis=0, keepdims=True)
    s1h1 = jnp.sum(st[NUM_TILES:, :HH], axis=0, keepdims=True)
    s2h1 = jnp.sum(st[NUM_TILES:, HH:], axis=0, keepdims=True)
    inv_e = 1.0 / float(E)
    m0, m1 = s1h0 * inv_e, s1h1 * inv_e
    v0 = s2h0 * inv_e - m0 * m0
    v1 = s2h1 * inv_e - m1 * m1
    r0 = (enew[0] - m0) * lax.rsqrt(v0 + 1e-5) * g[:, :HH] + b[:, :HH]
    r1 = (enew[1] - m1) * lax.rsqrt(v1 + 1e-5) * g[:, HH:] + b[:, HH:]
    r0 = jnp.maximum(r0, 0.0)
    r1 = jnp.maximum(r1, 0.0)
    ce = (jnp.dot(1.0 / e[...], wf[...], preferred_element_type=F32) + bf[...]
          + jnp.dot(r0, cW[:HH, :], preferred_element_type=F32)
          + jnp.dot(r1, cW[HH:, :], preferred_element_type=F32))
    out[0] = ce[:, :HH]
    out[1] = ce[:, HH:]


def _edge_linear2(e, enew, stats, wf, bf, cW, g, b):
    return pl.pallas_call(
        _edge_lin2_body,
        grid=(E // EBLK,),
        in_specs=[
            pl.BlockSpec((EBLK, 16), lambda i: (i, 0)),
            pl.BlockSpec((2, EBLK, HH), lambda i: (0, i, 0)),
            pl.BlockSpec((2 * NUM_TILES, 8, H), lambda i: (0, 0, 0)),
            pl.BlockSpec((16, H), lambda i: (0, 0)),
            pl.BlockSpec((1, H), lambda i: (0, 0)),
            pl.BlockSpec((H, H), lambda i: (0, 0)),
            pl.BlockSpec((1, H), lambda i: (0, 0)),
            pl.BlockSpec((1, H), lambda i: (0, 0)),
        ],
        out_specs=pl.BlockSpec((2, EBLK, HH), lambda i: (0, i, 0)),
        out_shape=_f32((2, E, HH)),
    )(e, enew, stats, wf, bf, cW, g, b)


def _node_upd_a_body(ah, acc, raw, ps, ps2):
    num = jnp.concatenate([acc[0, :, :HH], acc[1, :, :HH]], axis=1)
    den = jnp.concatenate([acc[0, :, HH:], acc[1, :, HH:]], axis=1)
    r = ah[...] + num / (den + 1e-6)
    raw[...] = r
    ps[...] = jnp.sum(r, axis=0, keepdims=True).reshape(1, 1, H)
    ps2[...] = jnp.sum(r * r, axis=0, keepdims=True).reshape(1, 1, H)


def _node_update_a(ah, acc):
    return pl.pallas_call(
        _node_upd_a_body,
        grid=(N // NBLK,),
        in_specs=[
            pl.BlockSpec((NBLK, H), lambda i: (i, 0)),
            pl.BlockSpec((2, NBLK, H), lambda i: (0, i, 0)),
        ],
        out_specs=[
            pl.BlockSpec((NBLK, H), lambda i: (i, 0)),
            pl.BlockSpec((1, 1, H), lambda i: (i, 0, 0)),
            pl.BlockSpec((1, 1, H), lambda i: (i, 0, 0)),
        ],
        out_shape=[_f32((N, H)), _f32((N // NBLK, 1, H)), _f32((N // NBLK, 1, H))],
    )(ah, acc)


def _node_upd_b_body(raw, ps, ps2, g, b, hin, out, psh):
    inv_n = 1.0 / float(N)
    m = jnp.sum(ps[...], axis=0) * inv_n
    v = jnp.sum(ps2[...], axis=0) * inv_n - m * m
    hn = (raw[...] - m) * lax.rsqrt(v + 1e-5) * g[...] + b[...]
    r = hin[...] + jnp.maximum(hn, 0.0)
    out[...] = r
    psh[...] = jnp.sum(r, axis=0, keepdims=True).reshape(1, 1, H)


def _node_update_b(raw, ps, ps2, g, b, hin):
    return pl.pallas_call(
        _node_upd_b_body,
        grid=(N // NBLK,),
        in_specs=[
            pl.BlockSpec((NBLK, H), lambda i: (i, 0)),
            pl.BlockSpec((N // NBLK, 1, H), lambda i: (0, 0, 0)),
            pl.BlockSpec((N // NBLK, 1, H), lambda i: (0, 0, 0)),
            pl.BlockSpec((1, H), lambda i: (0, 0)),
            pl.BlockSpec((1, H), lambda i: (0, 0)),
            pl.BlockSpec((NBLK, H), lambda i: (i, 0)),
        ],
        out_specs=[
            pl.BlockSpec((NBLK, H), lambda i: (i, 0)),
            pl.BlockSpec((1, 1, H), lambda i: (i, 0, 0)),
        ],
        out_shape=[_f32((N, H)), _f32((N // NBLK, 1, H))],
    )(raw, ps, ps2, g, b, hin)


def _head_body(psh, st, w1, b1, w2, b2, w3, b3, out):
    hm = jnp.sum(psh[...], axis=0) * (1.0 / float(N))
    z = jnp.concatenate([hm, st[...]], axis=1)
    z = jnp.maximum(jnp.dot(z, w1[...], preferred_element_type=F32) + b1[...], 0.0)
    z = jnp.maximum(jnp.dot(z, w2[...], preferred_element_type=F32) + b2[...], 0.0)
    out[...] = jnp.tanh(jnp.dot(z, w3[...], preferred_element_type=F32) + b3[...])


def _head(psh, st, w1, b1, w2, b2, w3, b3):
    return pl.pallas_call(_head_body, out_shape=_f32((1, 2)))(
        psh, st, w1, b1, w2, b2, w3, b3)


def _sigmoid16(x):
    return 1.0 / (1.0 + jnp.exp(-x))


def _sc_body(do_enew, *refs):
    if do_enew:
        (ce, dbt, eht, src, dst, accout, enew, bnstats,
         src_v0, src_v1, dst_v0, dst_v1, srca_v, ce_v0, ce_v1,
         db_v, eh_v, scat_v, stat_v, zero_v,
         s_src0, s_src1, s_dst0, s_dst1, s_ce0, s_ce1,
         s_gdb, s_geh, s_scat, s_en0, s_en1, acc_sh) = refs
        s_en = [s_en0, s_en1]
    else:
        (ce, dbt, eht, src, dst, accout,
         src_v0, src_v1, dst_v0, dst_v1, srca_v, ce_v0, ce_v1,
         db_v, eh_v, scat_v, stat_v, zero_v,
         s_src0, s_src1, s_dst0, s_dst1, s_ce0, s_ce1,
         s_gdb, s_geh, s_scat, acc_sh) = refs
    src_vs = [src_v0, src_v1]
    dst_vs = [dst_v0, dst_v1]
    ce_vs = [ce_v0, ce_v1]
    s_src = [s_src0, s_src1]
    s_dst = [s_dst0, s_dst1]
    s_ce = [s_ce0, s_ce1]

    cid = lax.axis_index("c")
    sid = lax.axis_index("s")

    def zloop(i, _):
        for c in range(H // 16):
            zero_v[i, pl.ds(c * 16, 16)] = jnp.zeros((16,), F32)
        return 0
    lax.fori_loop(0, ZROWS, zloop, 0)
    base = sid * ROWS_PER_TILE
    for k in range(ROWS_PER_TILE // ZROWS):
        pltpu.sync_copy(zero_v, acc_sh.at[pl.ds(base + k * ZROWS, ZROWS)])
    plsc.subcore_barrier()

    nblk = (NUM_EBLK - sid + NUM_TILES - 1) // NUM_TILES
    tbl_off = cid * N

    def _off(j):
        return (sid + j * NUM_TILES) * B

    def in_issue(slot, j):
        o = _off(j)
        pltpu.async_copy(src.at[pl.ds(o, B)], src_vs[slot], s_src[slot])
        pltpu.async_copy(dst.at[pl.ds(o, B)], dst_vs[slot], s_dst[slot])
        pltpu.async_copy(ce.at[pl.ds(cid * E + o, B)], ce_vs[slot], s_ce[slot])

    def in_wait(slot):
        pltpu.make_async_copy(src.at[pl.ds(0, B)], src_vs[slot], s_src[slot]).wait()
        pltpu.make_async_copy(dst.at[pl.ds(0, B)], dst_vs[slot], s_dst[slot]).wait()
        pltpu.make_async_copy(ce.at[pl.ds(0, B)], ce_vs[slot], s_ce[slot]).wait()

    def scat_wait():
        pltpu.make_async_copy(scat_v, acc_sh.at[dst_vs[0]], s_scat).wait()

    def enew_wait(slot):
        pltpu.make_async_copy(ce_vs[slot], enew.at[pl.ds(0, B)], s_en[slot]).wait()

    def block(j, slot, stats):
        in_wait(slot)

        @pl.when(j >= 1)
        def _():
            scat_wait()
        if do_enew:
            @pl.when(j >= 1)
            def _():
                enew_wait(1 - slot)

        @pl.when(j + 1 < nblk)
        def _():
            in_issue(1 - slot, j + 1)

        for c in range(B // 16):
            s16 = src_vs[slot][pl.ds(c * 16, 16)]
            srca_v[pl.ds(c * 16, 16)] = s16 + tbl_off
        d_db = pltpu.async_copy(dbt.at[srca_v], db_v, s_gdb)
        d_eh = pltpu.async_copy(eht.at[dst_vs[slot]], eh_v, s_geh)
        d_db.wait()
        d_eh.wait()

        def edge(e_i, st):
            st = list(st)
            for c in range(HH // 16):
                ds = pl.ds(c * 16, 16)
                en = (ce_vs[slot][e_i, ds] + db_v[e_i, ds]
                      + eh_v[e_i, pl.ds(cid * HH + c * 16, 16)])
                if do_enew:
                    ce_vs[slot][e_i, ds] = en
                    st[c] = st[c] + en
                    st[4 + c] = st[4 + c] + en * en
                sg = _sigmoid16(en)
                scat_v[e_i, ds] = sg * db_v[e_i, pl.ds(HH + c * 16, 16)]
                scat_v[e_i, pl.ds(HH + c * 16, 16)] = sg
            return tuple(st)

        stats = lax.fori_loop(0, B, edge, stats, unroll=8)
        if do_enew:
            pltpu.async_copy(ce_vs[slot], enew.at[pl.ds(cid * E + _off(j), B)],
                             s_en[slot])
        pltpu.async_copy(scat_v, acc_sh.at[dst_vs[slot]], s_scat, add=True)
        return stats

    in_issue(0, 0)
    zstats = tuple(jnp.zeros((16,), F32) for _ in range(8))
    npair = nblk // 2

    def pair(g, stats):
        stats = block(2 * g, 0, stats)
        return block(2 * g + 1, 1, stats)

    stats = lax.fori_loop(0, npair, pair, zstats)
    stats = lax.fori_loop(2 * npair, nblk, lambda j, st: block(j, 0, st), stats)

    scat_wait()
    if do_enew:
        @pl.when(nblk % 2 == 1)
        def _():
            enew_wait(0)

        @pl.when(nblk % 2 == 0)
        def _():
            enew_wait(1)
        for c in range(8):
            stat_v[0, 0, pl.ds(c * 16, 16)] = stats[c]
        pltpu.sync_copy(stat_v, bnstats.at[pl.ds(cid * NUM_TILES + sid, 1)])

    plsc.subcore_barrier()
    pltpu.sync_copy(
        acc_sh.at[pl.ds(base, ROWS_PER_TILE)],
        accout.at[pl.ds(cid * NPAD + base, ROWS_PER_TILE)])


def _make_sc_edge_pass(do_enew):
    out_type = [_f32((2 * NPAD, H))]
    if do_enew:
        out_type += [_f32((2 * E, HH)), _f32((2 * NUM_TILES, 8, H))]
    scratch = [
        pltpu.VMEM((B,), jnp.int32),
        pltpu.VMEM((B,), jnp.int32),
        pltpu.VMEM((B,), jnp.int32),
        pltpu.VMEM((B,), jnp.int32),
        pltpu.VMEM((B,), jnp.int32),
        pltpu.VMEM((B, HH), F32),
        pltpu.VMEM((B, HH), F32),
        pltpu.VMEM((B, H), F32),
        pltpu.VMEM((B, H), F32),
        pltpu.VMEM((B, H), F32),
        pltpu.VMEM((1, 8, H), F32),
        pltpu.VMEM((ZROWS, H), F32),
        pltpu.SemaphoreType.DMA,
        pltpu.SemaphoreType.DMA,
        pltpu.SemaphoreType.DMA,
        pltpu.SemaphoreType.DMA,
        pltpu.SemaphoreType.DMA,
        pltpu.SemaphoreType.DMA,
        pltpu.SemaphoreType.DMA,
        pltpu.SemaphoreType.DMA,
        pltpu.SemaphoreType.DMA,
    ]
    if do_enew:
        scratch += [pltpu.SemaphoreType.DMA, pltpu.SemaphoreType.DMA]
    scratch += [pltpu.VMEM_SHARED((NPAD, H), F32)]
    return pl.kernel(
        functools.partial(_sc_body, do_enew),
        out_type=out_type,
        mesh=plsc.VectorSubcoreMesh(core_axis_name="c", subcore_axis_name="s"),
        scratch_types=scratch,
    )


_sc_pass1 = _make_sc_edge_pass(True)
_sc_pass2 = _make_sc_edge_pass(False)


def kernel(h, e, state, params, edge_index):
    p = params
    l1, l2 = p["layers"]
    r2 = lambda b: b.reshape(1, H)
    src = edge_index[0]
    dst = edge_index[1]

    wf1, bf1, wf2, bf2 = _fold_weights(
        p["emb_e_W"], p["emb_e_b"].reshape(1, H),
        l1["C_W"], r2(l1["C_b"]), l2["C_W"], r2(l2["C_b"]))

    hh = _embed(h, p["emb_h_W"], p["emb_h_b"].reshape(1, H))

    ah, dbt, eht = _node_matmuls(
        hh, l1["A_W"], r2(l1["A_b"]), l1["B_W"], r2(l1["B_b"]),
        l1["D_W"], r2(l1["D_b"]), l1["E_W"], r2(l1["E_b"]))
    ce1 = _edge_linear1(e, wf1, bf1)
    acc1, enew1, stats1 = _sc_pass1(
        ce1.reshape(2 * E, HH), dbt.reshape(2 * N, H), eht, src, dst)
    raw1, ps1, ps21 = _node_update_a(ah, acc1.reshape(2, NPAD, H))
    hh1, _ = _node_update_b(raw1, ps1, ps21, r2(l1["bn_h_g"]),
                            r2(l1["bn_h_b"]), hh)

    ah2, dbt2, eht2 = _node_matmuls(
        hh1, l2["A_W"], r2(l2["A_b"]), l2["B_W"], r2(l2["B_b"]),
        l2["D_W"], r2(l2["D_b"]), l2["E_W"], r2(l2["E_b"]))
    ce2 = _edge_linear2(e, enew1.reshape(2, E, HH), stats1, wf2, bf2,
                        l2["C_W"], r2(l1["bn_e_g"]), r2(l1["bn_e_b"]))
    (acc2,) = _sc_pass2(
        ce2.reshape(2 * E, HH), dbt2.reshape(2 * N, H), eht2, src, dst)
    raw2, ps2, ps22 = _node_update_a(ah2, acc2.reshape(2, NPAD, H))
    hh2, psh = _node_update_b(raw2, ps2, ps22, r2(l2["bn_h_g"]),
                              r2(l2["bn_h_b"]), hh1)

    return _head(psh, state, p["l1_W"], p["l1_b"].reshape(1, 256),
                 p["l2_W"], p["l2_b"].reshape(1, 256),
                 p["l3_W"], p["l3_b"].reshape(1, 2))


# speedup vs baseline: 1.3945x; 1.3945x over previous
import functools

import jax
import jax.numpy as jnp
from jax import lax
from jax.experimental import pallas as pl
from jax.experimental.pallas import tpu as pltpu
from jax.experimental.pallas import tpu_sc as plsc

N = 10000
E = 320000
H = 128
HH = 64
NBLK = 1000
EBLK = 2000
B = 64
NUM_TILES = 16
NPAD = 10240
ROWS_PER_TILE = NPAD // NUM_TILES
ZROWS = 8
NUM_EBLK = E // B
F32 = jnp.float32


def _f32(x):
    return jax.ShapeDtypeStruct(x, F32)


def _fold_body(embW, embb, c1W, c1b, c2W, c2b, wf1, bf1, wf2, bf2):
    wf1[...] = jnp.dot(embW[...], c1W[...], preferred_element_type=F32)
    bf1[...] = jnp.dot(embb[...], c1W[...], preferred_element_type=F32) + c1b[...]
    wf2[...] = jnp.dot(embW[...], c2W[...], preferred_element_type=F32)
    bf2[...] = jnp.dot(embb[...], c2W[...], preferred_element_type=F32) + c2b[...]


def _fold_weights(embW, embb, c1W, c1b, c2W, c2b):
    return pl.pallas_call(
        _fold_body,
        out_shape=[_f32((16, H)), _f32((1, H)), _f32((16, H)), _f32((1, H))],
    )(embW, embb, c1W, c1b, c2W, c2b)


def _embed_body(h, w, b, out):
    out[...] = jnp.dot(h[...], w[...], preferred_element_type=F32) + b[...]


def _embed(h, w, b):
    return pl.pallas_call(
        _embed_body,
        grid=(N // NBLK,),
        in_specs=[
            pl.BlockSpec((NBLK, H), lambda i: (i, 0)),
            pl.BlockSpec((H, H), lambda i: (0, 0)),
            pl.BlockSpec((1, H), lambda i: (0, 0)),
        ],
        out_specs=pl.BlockSpec((NBLK, H), lambda i: (i, 0)),
        out_shape=_f32((N, H)),
    )(h, w, b)


def _node_mm_body(hh, aW, ab, bW, bb, dW, db_, eW, eb, ah, dbt, eht):
    x = hh[...]
    ah[...] = jnp.dot(x, aW[...], preferred_element_type=F32) + ab[...]
    Bh = jnp.dot(x, bW[...], preferred_element_type=F32) + bb[...]
    Dh = jnp.dot(x, dW[...], preferred_element_type=F32) + db_[...]
    Eh = jnp.dot(x, eW[...], preferred_element_type=F32) + eb[...]
    dbt[0] = jnp.concatenate([Dh[:, :HH], Bh[:, :HH]], axis=1)
    dbt[1] = jnp.concatenate([Dh[:, HH:], Bh[:, HH:]], axis=1)
    eht[...] = Eh


def _node_matmuls(hh, aW, ab, bW, bb, dW, db_, eW, eb):
    return pl.pallas_call(
        _node_mm_body,
        grid=(N // NBLK,),
        in_specs=[pl.BlockSpec((NBLK, H), lambda i: (i, 0))]
        + [pl.BlockSpec((H, H), lambda i: (0, 0)),
           pl.BlockSpec((1, H), lambda i: (0, 0))] * 4,
        out_specs=[
            pl.BlockSpec((NBLK, H), lambda i: (i, 0)),
            pl.BlockSpec((2, NBLK, H), lambda i: (0, i, 0)),
            pl.BlockSpec((NBLK, H), lambda i: (i, 0)),
        ],
        out_shape=[_f32((N, H)), _f32((2, N, H)), _f32((N, H))],
    )(hh, aW, ab, bW, bb, dW, db_, eW, eb)


def _edge_lin1_body(e, wf, bf, out):
    ce = jnp.dot(1.0 / e[...], wf[...], preferred_element_type=F32) + bf[...]
    out[0] = ce[:, :HH]
    out[1] = ce[:, HH:]


def _edge_linear1(e, wf, bf):
    return pl.pallas_call(
        _edge_lin1_body,
        grid=(E // EBLK,),
        in_specs=[
            pl.BlockSpec((EBLK, 16), lambda i: (i, 0)),
            pl.BlockSpec((16, H), lambda i: (0, 0)),
            pl.BlockSpec((1, H), lambda i: (0, 0)),
        ],
        out_specs=pl.BlockSpec((2, EBLK, HH), lambda i: (0, i, 0)),
        out_shape=_f32((2, E, HH)),
    )(e, wf, bf)


def _edge_lin2_body(e, enew, stats, wf, bf, cW, g, b, out):
    st = stats[:, 0, :]
    s1h0 = jnp.sum(st[:NUM_TILES, :HH], axis=0, keepdims=True)
    s2h0 = jnp.sum(st[:NUM_TILES, HH:], axis=0, keepdims=True)
    s1h1 = jnp.sum(st[NUM_TILES:, :HH], axis=0, keepdims=True)
    s2h1 = jnp.sum(st[NUM_TILES:, HH:], axis=0, keepdims=True)
    inv_e = 1.0 / float(E)
    m0, m1 = s1h0 * inv_e, s1h1 * inv_e
    v0 = s2h0 * inv_e - m0 * m0
    v1 = s2h1 * inv_e - m1 * m1
    r0 = (enew[0] - m0) * lax.rsqrt(v0 + 1e-5) * g[:, :HH] + b[:, :HH]
    r1 = (enew[1] - m1) * lax.rsqrt(v1 + 1e-5) * g[:, HH:] + b[:, HH:]
    r0 = jnp.maximum(r0, 0.0)
    r1 = jnp.maximum(r1, 0.0)
    ce = (jnp.dot(1.0 / e[...], wf[...], preferred_element_type=F32) + bf[...]
          + jnp.dot(r0, cW[:HH, :], preferred_element_type=F32)
          + jnp.dot(r1, cW[HH:, :], preferred_element_type=F32))
    out[0] = ce[:, :HH]
    out[1] = ce[:, HH:]


def _edge_linear2(e, enew, stats, wf, bf, cW, g, b):
    return pl.pallas_call(
        _edge_lin2_body,
        grid=(E // EBLK,),
        in_specs=[
            pl.BlockSpec((EBLK, 16), lambda i: (i, 0)),
            pl.BlockSpec((2, EBLK, HH), lambda i: (0, i, 0)),
            pl.BlockSpec((2 * NUM_TILES, 8, H), lambda i: (0, 0, 0)),
            pl.BlockSpec((16, H), lambda i: (0, 0)),
            pl.BlockSpec((1, H), lambda i: (0, 0)),
            pl.BlockSpec((H, H), lambda i: (0, 0)),
            pl.BlockSpec((1, H), lambda i: (0, 0)),
            pl.BlockSpec((1, H), lambda i: (0, 0)),
        ],
        out_specs=pl.BlockSpec((2, EBLK, HH), lambda i: (0, i, 0)),
        out_shape=_f32((2, E, HH)),
    )(e, enew, stats, wf, bf, cW, g, b)


def _node_upd_a_body(ah, acc, raw, ps, ps2):
    num = jnp.concatenate([acc[0, :, :HH], acc[1, :, :HH]], axis=1)
    den = jnp.concatenate([acc[0, :, HH:], acc[1, :, HH:]], axis=1)
    r = ah[...] + num / (den + 1e-6)
    raw[...] = r
    ps[...] = jnp.sum(r, axis=0, keepdims=True).reshape(1, 1, H)
    ps2[...] = jnp.sum(r * r, axis=0, keepdims=True).reshape(1, 1, H)


def _node_update_a(ah, acc):
    return pl.pallas_call(
        _node_upd_a_body,
        grid=(N // NBLK,),
        in_specs=[
            pl.BlockSpec((NBLK, H), lambda i: (i, 0)),
            pl.BlockSpec((2, NBLK, H), lambda i: (0, i, 0)),
        ],
        out_specs=[
            pl.BlockSpec((NBLK, H), lambda i: (i, 0)),
            pl.BlockSpec((1, 1, H), lambda i: (i, 0, 0)),
            pl.BlockSpec((1, 1, H), lambda i: (i, 0, 0)),
        ],
        out_shape=[_f32((N, H)), _f32((N // NBLK, 1, H)), _f32((N // NBLK, 1, H))],
    )(ah, acc)


def _node_upd_b_body(raw, ps, ps2, g, b, hin, out, psh):
    inv_n = 1.0 / float(N)
    m = jnp.sum(ps[...], axis=0) * inv_n
    v = jnp.sum(ps2[...], axis=0) * inv_n - m * m
    hn = (raw[...] - m) * lax.rsqrt(v + 1e-5) * g[...] + b[...]
    r = hin[...] + jnp.maximum(hn, 0.0)
    out[...] = r
    psh[...] = jnp.sum(r, axis=0, keepdims=True).reshape(1, 1, H)


def _node_update_b(raw, ps, ps2, g, b, hin):
    return pl.pallas_call(
        _node_upd_b_body,
        grid=(N // NBLK,),
        in_specs=[
            pl.BlockSpec((NBLK, H), lambda i: (i, 0)),
            pl.BlockSpec((N // NBLK, 1, H), lambda i: (0, 0, 0)),
            pl.BlockSpec((N // NBLK, 1, H), lambda i: (0, 0, 0)),
            pl.BlockSpec((1, H), lambda i: (0, 0)),
            pl.BlockSpec((1, H), lambda i: (0, 0)),
            pl.BlockSpec((NBLK, H), lambda i: (i, 0)),
        ],
        out_specs=[
            pl.BlockSpec((NBLK, H), lambda i: (i, 0)),
            pl.BlockSpec((1, 1, H), lambda i: (i, 0, 0)),
        ],
        out_shape=[_f32((N, H)), _f32((N // NBLK, 1, H))],
    )(raw, ps, ps2, g, b, hin)


def _head_body(psh, st, w1, b1, w2, b2, w3, b3, out):
    hm = jnp.sum(psh[...], axis=0) * (1.0 / float(N))
    z = jnp.concatenate([hm, st[...]], axis=1)
    z = jnp.maximum(jnp.dot(z, w1[...], preferred_element_type=F32) + b1[...], 0.0)
    z = jnp.maximum(jnp.dot(z, w2[...], preferred_element_type=F32) + b2[...], 0.0)
    out[...] = jnp.tanh(jnp.dot(z, w3[...], preferred_element_type=F32) + b3[...])


def _head(psh, st, w1, b1, w2, b2, w3, b3):
    return pl.pallas_call(_head_body, out_shape=_f32((1, 2)))(
        psh, st, w1, b1, w2, b2, w3, b3)


def _sigmoid16(x):
    return 1.0 / (1.0 + jnp.exp(-x))


def _sc_body(do_enew, *refs):
    if do_enew:
        (ce, dbt, eht, src, dst, accout, enew, bnstats,
         src_v0, src_v1, dst_v0, dst_v1, srca_v, ce_v0, ce_v1,
         db_v, eh_v, scat_v, stat_v, zero_v,
         s_src0, s_src1, s_dst0, s_dst1, s_ce0, s_ce1,
         s_gdb, s_geh, s_scat, s_en0, s_en1, acc_sh) = refs
        s_en = [s_en0, s_en1]
    else:
        (ce, dbt, eht, src, dst, accout,
         src_v0, src_v1, dst_v0, dst_v1, srca_v, ce_v0, ce_v1,
         db_v, eh_v, scat_v, stat_v, zero_v,
         s_src0, s_src1, s_dst0, s_dst1, s_ce0, s_ce1,
         s_gdb, s_geh, s_scat, acc_sh) = refs
    src_vs = [src_v0, src_v1]
    dst_vs = [dst_v0, dst_v1]
    ce_vs = [ce_v0, ce_v1]
    s_src = [s_src0, s_src1]
    s_dst = [s_dst0, s_dst1]
    s_ce = [s_ce0, s_ce1]

    cid = lax.axis_index("c")
    sid = lax.axis_index("s")

    def zloop(i, _):
        for c in range(H // 16):
            zero_v[i, pl.ds(c * 16, 16)] = jnp.zeros((16,), F32)
        return 0
    lax.fori_loop(0, ZROWS, zloop, 0)
    base = sid * ROWS_PER_TILE
    for k in range(ROWS_PER_TILE // ZROWS):
        pltpu.sync_copy(zero_v, acc_sh.at[pl.ds(base + k * ZROWS, ZROWS)])
    plsc.subcore_barrier()

    nblk = (NUM_EBLK - sid + NUM_TILES - 1) // NUM_TILES
    tbl_off = cid * N

    def _off(j):
        return (sid + j * NUM_TILES) * B

    def in_issue(slot, j):
        o = _off(j)
        pltpu.async_copy(src.at[pl.ds(o, B)], src_vs[slot], s_src[slot])
        pltpu.async_copy(dst.at[pl.ds(o, B)], dst_vs[slot], s_dst[slot])
        pltpu.async_copy(ce.at[pl.ds(cid * E + o, B)], ce_vs[slot], s_ce[slot])

    def in_wait(slot):
        pltpu.make_async_copy(src.at[pl.ds(0, B)], src_vs[slot], s_src[slot]).wait()
        pltpu.make_async_copy(dst.at[pl.ds(0, B)], dst_vs[slot], s_dst[slot]).wait()
        pltpu.make_async_copy(ce.at[pl.ds(0, B)], ce_vs[slot], s_ce[slot]).wait()

    def scat_wait():
        pltpu.make_async_copy(scat_v, acc_sh.at[dst_vs[0]], s_scat).wait()

    def enew_wait(slot):
        pltpu.make_async_copy(ce_vs[slot], enew.at[pl.ds(0, B)], s_en[slot]).wait()

    def block(j, slot, stats):
        in_wait(slot)

        @pl.when(j >= 1)
        def _():
            scat_wait()
        if do_enew:
            @pl.when(j >= 1)
            def _():
                enew_wait(1 - slot)

        @pl.when(j + 1 < nblk)
        def _():
            in_issue(1 - slot, j + 1)

        for c in range(B // 16):
            s16 = src_vs[slot][pl.ds(c * 16, 16)]
            srca_v[pl.ds(c * 16, 16)] = s16 + tbl_off
        d_db = pltpu.async_copy(dbt.at[srca_v], db_v, s_gdb)
        d_eh = pltpu.async_copy(eht.at[dst_vs[slot]], eh_v, s_geh)
        d_db.wait()
        d_eh.wait()

        def edge(e_i, st):
            st = list(st)
            for c in range(HH // 16):
                ds = pl.ds(c * 16, 16)
                en = (ce_vs[slot][e_i, ds] + db_v[e_i, ds]
                      + eh_v[e_i, pl.ds(cid * HH + c * 16, 16)])
                if do_enew:
                    ce_vs[slot][e_i, ds] = en
                    st[c] = st[c] + en
                    st[4 + c] = st[4 + c] + en * en
                sg = _sigmoid16(en)
                scat_v[e_i, ds] = sg * db_v[e_i, pl.ds(HH + c * 16, 16)]
                scat_v[e_i, pl.ds(HH + c * 16, 16)] = sg
            return tuple(st)

        stats = plsc.parallel_loop(0, B, 1, unroll=4, carry=tuple(stats))(edge)
        if do_enew:
            pltpu.async_copy(ce_vs[slot], enew.at[pl.ds(cid * E + _off(j), B)],
                             s_en[slot])
        pltpu.async_copy(scat_v, acc_sh.at[dst_vs[slot]], s_scat, add=True)
        return stats

    in_issue(0, 0)
    zstats = tuple(jnp.zeros((16,), F32) for _ in range(8))
    npair = nblk // 2

    def pair(g, stats):
        stats = block(2 * g, 0, stats)
        return block(2 * g + 1, 1, stats)

    stats = lax.fori_loop(0, npair, pair, zstats)
    stats = lax.fori_loop(2 * npair, nblk, lambda j, st: block(j, 0, st), stats)

    scat_wait()
    if do_enew:
        @pl.when(nblk % 2 == 1)
        def _():
            enew_wait(0)

        @pl.when(nblk % 2 == 0)
        def _():
            enew_wait(1)
        for c in range(8):
            stat_v[0, 0, pl.ds(c * 16, 16)] = stats[c]
        pltpu.sync_copy(stat_v, bnstats.at[pl.ds(cid * NUM_TILES + sid, 1)])

    plsc.subcore_barrier()
    pltpu.sync_copy(
        acc_sh.at[pl.ds(base, ROWS_PER_TILE)],
        accout.at[pl.ds(cid * NPAD + base, ROWS_PER_TILE)])


def _make_sc_edge_pass(do_enew):
    out_type = [_f32((2 * NPAD, H))]
    if do_enew:
        out_type += [_f32((2 * E, HH)), _f32((2 * NUM_TILES, 8, H))]
    scratch = [
        pltpu.VMEM((B,), jnp.int32),
        pltpu.VMEM((B,), jnp.int32),
        pltpu.VMEM((B,), jnp.int32),
        pltpu.VMEM((B,), jnp.int32),
        pltpu.VMEM((B,), jnp.int32),
        pltpu.VMEM((B, HH), F32),
        pltpu.VMEM((B, HH), F32),
        pltpu.VMEM((B, H), F32),
        pltpu.VMEM((B, H), F32),
        pltpu.VMEM((B, H), F32),
        pltpu.VMEM((1, 8, H), F32),
        pltpu.VMEM((ZROWS, H), F32),
        pltpu.SemaphoreType.DMA,
        pltpu.SemaphoreType.DMA,
        pltpu.SemaphoreType.DMA,
        pltpu.SemaphoreType.DMA,
        pltpu.SemaphoreType.DMA,
        pltpu.SemaphoreType.DMA,
        pltpu.SemaphoreType.DMA,
        pltpu.SemaphoreType.DMA,
        pltpu.SemaphoreType.DMA,
    ]
    if do_enew:
        scratch += [pltpu.SemaphoreType.DMA, pltpu.SemaphoreType.DMA]
    scratch += [pltpu.VMEM_SHARED((NPAD, H), F32)]
    return pl.kernel(
        functools.partial(_sc_body, do_enew),
        out_type=out_type,
        mesh=plsc.VectorSubcoreMesh(core_axis_name="c", subcore_axis_name="s"),
        scratch_types=scratch,
    )


_sc_pass1 = _make_sc_edge_pass(True)
_sc_pass2 = _make_sc_edge_pass(False)


def kernel(h, e, state, params, edge_index):
    p = params
    l1, l2 = p["layers"]
    r2 = lambda b: b.reshape(1, H)
    src = edge_index[0]
    dst = edge_index[1]

    wf1, bf1, wf2, bf2 = _fold_weights(
        p["emb_e_W"], p["emb_e_b"].reshape(1, H),
        l1["C_W"], r2(l1["C_b"]), l2["C_W"], r2(l2["C_b"]))

    hh = _embed(h, p["emb_h_W"], p["emb_h_b"].reshape(1, H))

    ah, dbt, eht = _node_matmuls(
        hh, l1["A_W"], r2(l1["A_b"]), l1["B_W"], r2(l1["B_b"]),
        l1["D_W"], r2(l1["D_b"]), l1["E_W"], r2(l1["E_b"]))
    ce1 = _edge_linear1(e, wf1, bf1)
    acc1, enew1, stats1 = _sc_pass1(
        ce1.reshape(2 * E, HH), dbt.reshape(2 * N, H), eht, src, dst)
    raw1, ps1, ps21 = _node_update_a(ah, acc1.reshape(2, NPAD, H))
    hh1, _ = _node_update_b(raw1, ps1, ps21, r2(l1["bn_h_g"]),
                            r2(l1["bn_h_b"]), hh)

    ah2, dbt2, eht2 = _node_matmuls(
        hh1, l2["A_W"], r2(l2["A_b"]), l2["B_W"], r2(l2["B_b"]),
        l2["D_W"], r2(l2["D_b"]), l2["E_W"], r2(l2["E_b"]))
    ce2 = _edge_linear2(e, enew1.reshape(2, E, HH), stats1, wf2, bf2,
                        l2["C_W"], r2(l1["bn_e_g"]), r2(l1["bn_e_b"]))
    (acc2,) = _sc_pass2(
        ce2.reshape(2 * E, HH), dbt2.reshape(2 * N, H), eht2, src, dst)
    raw2, ps2, ps22 = _node_update_a(ah2, acc2.reshape(2, NPAD, H))
    hh2, psh = _node_update_b(raw2, ps2, ps22, r2(l2["bn_h_g"]),
                              r2(l2["bn_h_b"]), hh1)

    return _head(psh, state, p["l1_W"], p["l1_b"].reshape(1, 256),
                 p["l2_W"], p["l2_b"].reshape(1, 256),
                 p["l3_W"], p["l3_b"].reshape(1, 2))


# speedup vs baseline: 1.3974x; 1.0021x over previous
import functools

import jax
import jax.numpy as jnp
from jax import lax
from jax.experimental import pallas as pl
from jax.experimental.pallas import tpu as pltpu
from jax.experimental.pallas import tpu_sc as plsc

N = 10000
E = 320000
H = 128
HH = 64
NBLK = 1000
EBLK = 2000
B = 64
NUM_TILES = 16
NPAD = 10240
ROWS_PER_TILE = NPAD // NUM_TILES
ZROWS = 8
NUM_EBLK = E // B
F32 = jnp.float32


def _f32(x):
    return jax.ShapeDtypeStruct(x, F32)


def _fold_body(embW, embb, c1W, c1b, c2W, c2b, wf1, bf1, wf2, bf2):
    wf1[...] = jnp.dot(embW[...], c1W[...], preferred_element_type=F32)
    bf1[...] = jnp.dot(embb[...], c1W[...], preferred_element_type=F32) + c1b[...]
    wf2[...] = jnp.dot(embW[...], c2W[...], preferred_element_type=F32)
    bf2[...] = jnp.dot(embb[...], c2W[...], preferred_element_type=F32) + c2b[...]


def _fold_weights(embW, embb, c1W, c1b, c2W, c2b):
    return pl.pallas_call(
        _fold_body,
        out_shape=[_f32((16, H)), _f32((1, H)), _f32((16, H)), _f32((1, H))],
    )(embW, embb, c1W, c1b, c2W, c2b)


def _embed_body(h, w, b, out):
    out[...] = jnp.dot(h[...], w[...], preferred_element_type=F32) + b[...]


def _embed(h, w, b):
    return pl.pallas_call(
        _embed_body,
        grid=(N // NBLK,),
        in_specs=[
            pl.BlockSpec((NBLK, H), lambda i: (i, 0)),
            pl.BlockSpec((H, H), lambda i: (0, 0)),
            pl.BlockSpec((1, H), lambda i: (0, 0)),
        ],
        out_specs=pl.BlockSpec((NBLK, H), lambda i: (i, 0)),
        out_shape=_f32((N, H)),
    )(h, w, b)


def _node_mm_body(hh, aW, ab, bW, bb, dW, db_, eW, eb, ah, dbt, eht):
    x = hh[...]
    ah[...] = jnp.dot(x, aW[...], preferred_element_type=F32) + ab[...]
    Bh = jnp.dot(x, bW[...], preferred_element_type=F32) + bb[...]
    Dh = jnp.dot(x, dW[...], preferred_element_type=F32) + db_[...]
    Eh = jnp.dot(x, eW[...], preferred_element_type=F32) + eb[...]
    dbt[0] = jnp.concatenate([Dh[:, :HH], Bh[:, :HH]], axis=1)
    dbt[1] = jnp.concatenate([Dh[:, HH:], Bh[:, HH:]], axis=1)
    eht[...] = Eh


def _node_matmuls(hh, aW, ab, bW, bb, dW, db_, eW, eb):
    return pl.pallas_call(
        _node_mm_body,
        grid=(N // NBLK,),
        in_specs=[pl.BlockSpec((NBLK, H), lambda i: (i, 0))]
        + [pl.BlockSpec((H, H), lambda i: (0, 0)),
           pl.BlockSpec((1, H), lambda i: (0, 0))] * 4,
        out_specs=[
            pl.BlockSpec((NBLK, H), lambda i: (i, 0)),
            pl.BlockSpec((2, NBLK, H), lambda i: (0, i, 0)),
            pl.BlockSpec((NBLK, H), lambda i: (i, 0)),
        ],
        out_shape=[_f32((N, H)), _f32((2, N, H)), _f32((N, H))],
    )(hh, aW, ab, bW, bb, dW, db_, eW, eb)


def _edge_lin1_body(e, wf, bf, out):
    ce = jnp.dot(1.0 / e[...], wf[...], preferred_element_type=F32) + bf[...]
    out[0] = ce[:, :HH]
    out[1] = ce[:, HH:]


def _edge_linear1(e, wf, bf):
    return pl.pallas_call(
        _edge_lin1_body,
        grid=(E // EBLK,),
        in_specs=[
            pl.BlockSpec((EBLK, 16), lambda i: (i, 0)),
            pl.BlockSpec((16, H), lambda i: (0, 0)),
            pl.BlockSpec((1, H), lambda i: (0, 0)),
        ],
        out_specs=pl.BlockSpec((2, EBLK, HH), lambda i: (0, i, 0)),
        out_shape=_f32((2, E, HH)),
    )(e, wf, bf)


def _edge_lin2_body(e, enew, stats, wf, bf, cW, g, b, out):
    st = stats[:, 0, :]
    s1h0 = jnp.sum(st[:NUM_TILES, :HH], axis=0, keepdims=True)
    s2h0 = jnp.sum(st[:NUM_TILES, HH:], axis=0, keepdims=True)
    s1h1 = jnp.sum(st[NUM_TILES:, :HH], axis=0, keepdims=True)
    s2h1 = jnp.sum(st[NUM_TILES:, HH:], axis=0, keepdims=True)
    inv_e = 1.0 / float(E)
    m0, m1 = s1h0 * inv_e, s1h1 * inv_e
    v0 = s2h0 * inv_e - m0 * m0
    v1 = s2h1 * inv_e - m1 * m1
    r0 = (enew[0] - m0) * lax.rsqrt(v0 + 1e-5) * g[:, :HH] + b[:, :HH]
    r1 = (enew[1] - m1) * lax.rsqrt(v1 + 1e-5) * g[:, HH:] + b[:, HH:]
    r0 = jnp.maximum(r0, 0.0)
    r1 = jnp.maximum(r1, 0.0)
    ce = (jnp.dot(1.0 / e[...], wf[...], preferred_element_type=F32) + bf[...]
          + jnp.dot(r0, cW[:HH, :], preferred_element_type=F32)
          + jnp.dot(r1, cW[HH:, :], preferred_element_type=F32))
    out[0] = ce[:, :HH]
    out[1] = ce[:, HH:]


def _edge_linear2(e, enew, stats, wf, bf, cW, g, b):
    return pl.pallas_call(
        _edge_lin2_body,
        grid=(E // EBLK,),
        in_specs=[
            pl.BlockSpec((EBLK, 16), lambda i: (i, 0)),
            pl.BlockSpec((2, EBLK, HH), lambda i: (0, i, 0)),
            pl.BlockSpec((2 * NUM_TILES, 8, H), lambda i: (0, 0, 0)),
            pl.BlockSpec((16, H), lambda i: (0, 0)),
            pl.BlockSpec((1, H), lambda i: (0, 0)),
            pl.BlockSpec((H, H), lambda i: (0, 0)),
            pl.BlockSpec((1, H), lambda i: (0, 0)),
            pl.BlockSpec((1, H), lambda i: (0, 0)),
        ],
        out_specs=pl.BlockSpec((2, EBLK, HH), lambda i: (0, i, 0)),
        out_shape=_f32((2, E, HH)),
    )(e, enew, stats, wf, bf, cW, g, b)


def _node_upd_a_body(ah, acc, raw, ps, ps2):
    num = jnp.concatenate([acc[0, :, :HH], acc[1, :, :HH]], axis=1)
    den = jnp.concatenate([acc[0, :, HH:], acc[1, :, HH:]], axis=1)
    r = ah[...] + num / (den + 1e-6)
    raw[...] = r
    ps[...] = jnp.sum(r, axis=0, keepdims=True).reshape(1, 1, H)
    ps2[...] = jnp.sum(r * r, axis=0, keepdims=True).reshape(1, 1, H)


def _node_update_a(ah, acc):
    return pl.pallas_call(
        _node_upd_a_body,
        grid=(N // NBLK,),
        in_specs=[
            pl.BlockSpec((NBLK, H), lambda i: (i, 0)),
            pl.BlockSpec((2, NBLK, H), lambda i: (0, i, 0)),
        ],
        out_specs=[
            pl.BlockSpec((NBLK, H), lambda i: (i, 0)),
            pl.BlockSpec((1, 1, H), lambda i: (i, 0, 0)),
            pl.BlockSpec((1, 1, H), lambda i: (i, 0, 0)),
        ],
        out_shape=[_f32((N, H)), _f32((N // NBLK, 1, H)), _f32((N // NBLK, 1, H))],
    )(ah, acc)


def _node_upd_b_body(raw, ps, ps2, g, b, hin, out, psh):
    inv_n = 1.0 / float(N)
    m = jnp.sum(ps[...], axis=0) * inv_n
    v = jnp.sum(ps2[...], axis=0) * inv_n - m * m
    hn = (raw[...] - m) * lax.rsqrt(v + 1e-5) * g[...] + b[...]
    r = hin[...] + jnp.maximum(hn, 0.0)
    out[...] = r
    psh[...] = jnp.sum(r, axis=0, keepdims=True).reshape(1, 1, H)


def _node_update_b(raw, ps, ps2, g, b, hin):
    return pl.pallas_call(
        _node_upd_b_body,
        grid=(N // NBLK,),
        in_specs=[
            pl.BlockSpec((NBLK, H), lambda i: (i, 0)),
            pl.BlockSpec((N // NBLK, 1, H), lambda i: (0, 0, 0)),
            pl.BlockSpec((N // NBLK, 1, H), lambda i: (0, 0, 0)),
            pl.BlockSpec((1, H), lambda i: (0, 0)),
            pl.BlockSpec((1, H), lambda i: (0, 0)),
            pl.BlockSpec((NBLK, H), lambda i: (i, 0)),
        ],
        out_specs=[
            pl.BlockSpec((NBLK, H), lambda i: (i, 0)),
            pl.BlockSpec((1, 1, H), lambda i: (i, 0, 0)),
        ],
        out_shape=[_f32((N, H)), _f32((N // NBLK, 1, H))],
    )(raw, ps, ps2, g, b, hin)


def _head_body(psh, st, w1, b1, w2, b2, w3, b3, out):
    hm = jnp.sum(psh[...], axis=0) * (1.0 / float(N))
    z = jnp.concatenate([hm, st[...]], axis=1)
    z = jnp.maximum(jnp.dot(z, w1[...], preferred_element_type=F32) + b1[...], 0.0)
    z = jnp.maximum(jnp.dot(z, w2[...], preferred_element_type=F32) + b2[...], 0.0)
    out[...] = jnp.tanh(jnp.dot(z, w3[...], preferred_element_type=F32) + b3[...])


def _head(psh, st, w1, b1, w2, b2, w3, b3):
    return pl.pallas_call(_head_body, out_shape=_f32((1, 2)))(
        psh, st, w1, b1, w2, b2, w3, b3)


def _sigmoid16(x):
    return 1.0 / (1.0 + jnp.exp(-x))


def _sc_body(do_enew, *refs):
    if do_enew:
        (ce, dbt, eht, src, dst, accout, enew, bnstats,
         src_v0, src_v1, dst_v0, dst_v1, srca_v, ce_v0, ce_v1,
         db_v, eh_v, scat_v, stat_v, zero_v,
         s_src0, s_src1, s_dst0, s_dst1, s_ce0, s_ce1,
         s_gdb, s_geh, s_scat, s_en0, s_en1, acc_sh) = refs
        s_en = [s_en0, s_en1]
    else:
        (ce, dbt, eht, src, dst, accout,
         src_v0, src_v1, dst_v0, dst_v1, srca_v, ce_v0, ce_v1,
         db_v, eh_v, scat_v, stat_v, zero_v,
         s_src0, s_src1, s_dst0, s_dst1, s_ce0, s_ce1,
         s_gdb, s_geh, s_scat, acc_sh) = refs
    src_vs = [src_v0, src_v1]
    dst_vs = [dst_v0, dst_v1]
    ce_vs = [ce_v0, ce_v1]
    s_src = [s_src0, s_src1]
    s_dst = [s_dst0, s_dst1]
    s_ce = [s_ce0, s_ce1]

    cid = lax.axis_index("c")
    sid = lax.axis_index("s")

    def zloop(i, _):
        for c in range(H // 16):
            zero_v[i, pl.ds(c * 16, 16)] = jnp.zeros((16,), F32)
        return 0
    lax.fori_loop(0, ZROWS, zloop, 0)
    base = sid * ROWS_PER_TILE
    for k in range(ROWS_PER_TILE // ZROWS):
        pltpu.sync_copy(zero_v, acc_sh.at[pl.ds(base + k * ZROWS, ZROWS)])
    plsc.subcore_barrier()

    nblk = (NUM_EBLK - sid + NUM_TILES - 1) // NUM_TILES
    tbl_off = cid * N

    def _off(j):
        return (sid + j * NUM_TILES) * B

    def in_issue(slot, j):
        o = _off(j)
        pltpu.async_copy(src.at[pl.ds(o, B)], src_vs[slot], s_src[slot])
        pltpu.async_copy(dst.at[pl.ds(o, B)], dst_vs[slot], s_dst[slot])
        pltpu.async_copy(ce.at[pl.ds(cid * E + o, B)], ce_vs[slot], s_ce[slot])

    def in_wait(slot):
        pltpu.make_async_copy(src.at[pl.ds(0, B)], src_vs[slot], s_src[slot]).wait()
        pltpu.make_async_copy(dst.at[pl.ds(0, B)], dst_vs[slot], s_dst[slot]).wait()
        pltpu.make_async_copy(ce.at[pl.ds(0, B)], ce_vs[slot], s_ce[slot]).wait()

    def scat_wait():
        pltpu.make_async_copy(scat_v, acc_sh.at[dst_vs[0]], s_scat).wait()

    def enew_wait(slot):
        pltpu.make_async_copy(ce_vs[slot], enew.at[pl.ds(0, B)], s_en[slot]).wait()

    def block(j, slot, stats):
        in_wait(slot)

        @pl.when(j >= 1)
        def _():
            scat_wait()
        if do_enew:
            @pl.when(j >= 1)
            def _():
                enew_wait(1 - slot)

        @pl.when(j + 1 < nblk)
        def _():
            in_issue(1 - slot, j + 1)

        for c in range(B // 16):
            s16 = src_vs[slot][pl.ds(c * 16, 16)]
            srca_v[pl.ds(c * 16, 16)] = s16 + tbl_off
        d_db = pltpu.async_copy(dbt.at[srca_v], db_v, s_gdb)
        d_eh = pltpu.async_copy(eht.at[dst_vs[slot]], eh_v, s_geh)
        d_db.wait()
        d_eh.wait()

        def edge(e_i, st):
            st = list(st)
            for c in range(HH // 16):
                ds = pl.ds(c * 16, 16)
                en = (ce_vs[slot][e_i, ds] + db_v[e_i, ds]
                      + eh_v[e_i, pl.ds(cid * HH + c * 16, 16)])
                if do_enew:
                    ce_vs[slot][e_i, ds] = en
                    st[c] = st[c] + en
                    st[4 + c] = st[4 + c] + en * en
                sg = _sigmoid16(en)
                scat_v[e_i, ds] = sg * db_v[e_i, pl.ds(HH + c * 16, 16)]
                scat_v[e_i, pl.ds(HH + c * 16, 16)] = sg
            return tuple(st)

        stats = plsc.parallel_loop(0, B, 1, unroll=8, carry=tuple(stats))(edge)
        if do_enew:
            pltpu.async_copy(ce_vs[slot], enew.at[pl.ds(cid * E + _off(j), B)],
                             s_en[slot])
        pltpu.async_copy(scat_v, acc_sh.at[dst_vs[slot]], s_scat, add=True)
        return stats

    in_issue(0, 0)
    zstats = tuple(jnp.zeros((16,), F32) for _ in range(8))
    npair = nblk // 2

    def pair(g, stats):
        stats = block(2 * g, 0, stats)
        return block(2 * g + 1, 1, stats)

    stats = lax.fori_loop(0, npair, pair, zstats)
    stats = lax.fori_loop(2 * npair, nblk, lambda j, st: block(j, 0, st), stats)

    scat_wait()
    if do_enew:
        @pl.when(nblk % 2 == 1)
        def _():
            enew_wait(0)

        @pl.when(nblk % 2 == 0)
        def _():
            enew_wait(1)
        for c in range(8):
            stat_v[0, 0, pl.ds(c * 16, 16)] = stats[c]
        pltpu.sync_copy(stat_v, bnstats.at[pl.ds(cid * NUM_TILES + sid, 1)])

    plsc.subcore_barrier()
    pltpu.sync_copy(
        acc_sh.at[pl.ds(base, ROWS_PER_TILE)],
        accout.at[pl.ds(cid * NPAD + base, ROWS_PER_TILE)])


def _make_sc_edge_pass(do_enew):
    out_type = [_f32((2 * NPAD, H))]
    if do_enew:
        out_type += [_f32((2 * E, HH)), _f32((2 * NUM_TILES, 8, H))]
    scratch = [
        pltpu.VMEM((B,), jnp.int32),
        pltpu.VMEM((B,), jnp.int32),
        pltpu.VMEM((B,), jnp.int32),
        pltpu.VMEM((B,), jnp.int32),
        pltpu.VMEM((B,), jnp.int32),
        pltpu.VMEM((B, HH), F32),
        pltpu.VMEM((B, HH), F32),
        pltpu.VMEM((B, H), F32),
        pltpu.VMEM((B, H), F32),
        pltpu.VMEM((B, H), F32),
        pltpu.VMEM((1, 8, H), F32),
        pltpu.VMEM((ZROWS, H), F32),
        pltpu.SemaphoreType.DMA,
        pltpu.SemaphoreType.DMA,
        pltpu.SemaphoreType.DMA,
        pltpu.SemaphoreType.DMA,
        pltpu.SemaphoreType.DMA,
        pltpu.SemaphoreType.DMA,
        pltpu.SemaphoreType.DMA,
        pltpu.SemaphoreType.DMA,
        pltpu.SemaphoreType.DMA,
    ]
    if do_enew:
        scratch += [pltpu.SemaphoreType.DMA, pltpu.SemaphoreType.DMA]
    scratch += [pltpu.VMEM_SHARED((NPAD, H), F32)]
    return pl.kernel(
        functools.partial(_sc_body, do_enew),
        out_type=out_type,
        mesh=plsc.VectorSubcoreMesh(core_axis_name="c", subcore_axis_name="s"),
        scratch_types=scratch,
    )


_sc_pass1 = _make_sc_edge_pass(True)
_sc_pass2 = _make_sc_edge_pass(False)


def kernel(h, e, state, params, edge_index):
    p = params
    l1, l2 = p["layers"]
    r2 = lambda b: b.reshape(1, H)
    src = edge_index[0]
    dst = edge_index[1]

    wf1, bf1, wf2, bf2 = _fold_weights(
        p["emb_e_W"], p["emb_e_b"].reshape(1, H),
        l1["C_W"], r2(l1["C_b"]), l2["C_W"], r2(l2["C_b"]))

    hh = _embed(h, p["emb_h_W"], p["emb_h_b"].reshape(1, H))

    ah, dbt, eht = _node_matmuls(
        hh, l1["A_W"], r2(l1["A_b"]), l1["B_W"], r2(l1["B_b"]),
        l1["D_W"], r2(l1["D_b"]), l1["E_W"], r2(l1["E_b"]))
    ce1 = _edge_linear1(e, wf1, bf1)
    acc1, enew1, stats1 = _sc_pass1(
        ce1.reshape(2 * E, HH), dbt.reshape(2 * N, H), eht, src, dst)
    raw1, ps1, ps21 = _node_update_a(ah, acc1.reshape(2, NPAD, H))
    hh1, _ = _node_update_b(raw1, ps1, ps21, r2(l1["bn_h_g"]),
                            r2(l1["bn_h_b"]), hh)

    ah2, dbt2, eht2 = _node_matmuls(
        hh1, l2["A_W"], r2(l2["A_b"]), l2["B_W"], r2(l2["B_b"]),
        l2["D_W"], r2(l2["D_b"]), l2["E_W"], r2(l2["E_b"]))
    ce2 = _edge_linear2(e, enew1.reshape(2, E, HH), stats1, wf2, bf2,
                        l2["C_W"], r2(l1["bn_e_g"]), r2(l1["bn_e_b"]))
    (acc2,) = _sc_pass2(
        ce2.reshape(2 * E, HH), dbt2.reshape(2 * N, H), eht2, src, dst)
    raw2, ps2, ps22 = _node_update_a(ah2, acc2.reshape(2, NPAD, H))
    hh2, psh = _node_update_b(raw2, ps2, ps22, r2(l2["bn_h_g"]),
                              r2(l2["bn_h_b"]), hh1)

    return _head(psh, state, p["l1_W"], p["l1_b"].reshape(1, 256),
                 p["l2_W"], p["l2_b"].reshape(1, 256),
                 p["l3_W"], p["l3_b"].reshape(1, 2))


# speedup vs baseline: 1.7573x; 1.2575x over previous
import functools

import jax
import jax.numpy as jnp
from jax import lax
from jax.experimental import pallas as pl
from jax.experimental.pallas import tpu as pltpu
from jax.experimental.pallas import tpu_sc as plsc

N = 10000
E = 320000
H = 128
HH = 64
NBLK = 1000
EBLK = 2000
B = 64
NUM_TILES = 16
DRAIN_ROWS = 624
ZROWS = 16
NUM_EBLK = E // B
F32 = jnp.float32


def _f32(x):
    return jax.ShapeDtypeStruct(x, F32)


def _fold_body(embW, embb, c1W, c1b, c2W, c2b, wf1, bf1, wf2, bf2):
    wf1[...] = jnp.dot(embW[...], c1W[...], preferred_element_type=F32)
    bf1[...] = jnp.dot(embb[...], c1W[...], preferred_element_type=F32) + c1b[...]
    wf2[...] = jnp.dot(embW[...], c2W[...], preferred_element_type=F32)
    bf2[...] = jnp.dot(embb[...], c2W[...], preferred_element_type=F32) + c2b[...]


def _fold_weights(embW, embb, c1W, c1b, c2W, c2b):
    return pl.pallas_call(
        _fold_body,
        out_shape=[_f32((16, H)), _f32((1, H)), _f32((16, H)), _f32((1, H))],
    )(embW, embb, c1W, c1b, c2W, c2b)


def _embed_body(h, w, b, out):
    out[...] = jnp.dot(h[...], w[...], preferred_element_type=F32) + b[...]


def _embed(h, w, b):
    return pl.pallas_call(
        _embed_body,
        grid=(N // NBLK,),
        in_specs=[
            pl.BlockSpec((NBLK, H), lambda i: (i, 0)),
            pl.BlockSpec((H, H), lambda i: (0, 0)),
            pl.BlockSpec((1, H), lambda i: (0, 0)),
        ],
        out_specs=pl.BlockSpec((NBLK, H), lambda i: (i, 0)),
        out_shape=_f32((N, H)),
    )(h, w, b)


def _node_mm_body(hh, aW, ab, bW, bb, dW, db_, eW, eb, ah, dbt, eht):
    x = hh[...]
    ah[...] = jnp.dot(x, aW[...], preferred_element_type=F32) + ab[...]
    Bh = jnp.dot(x, bW[...], preferred_element_type=F32) + bb[...]
    Dh = jnp.dot(x, dW[...], preferred_element_type=F32) + db_[...]
    Eh = jnp.dot(x, eW[...], preferred_element_type=F32) + eb[...]
    dbt[0] = jnp.concatenate([Dh[:, :HH], Bh[:, :HH]], axis=1)
    dbt[1] = jnp.concatenate([Dh[:, HH:], Bh[:, HH:]], axis=1)
    eht[...] = Eh


def _node_matmuls(hh, aW, ab, bW, bb, dW, db_, eW, eb):
    return pl.pallas_call(
        _node_mm_body,
        grid=(N // NBLK,),
        in_specs=[pl.BlockSpec((NBLK, H), lambda i: (i, 0))]
        + [pl.BlockSpec((H, H), lambda i: (0, 0)),
           pl.BlockSpec((1, H), lambda i: (0, 0))] * 4,
        out_specs=[
            pl.BlockSpec((NBLK, H), lambda i: (i, 0)),
            pl.BlockSpec((2, NBLK, H), lambda i: (0, i, 0)),
            pl.BlockSpec((NBLK, H), lambda i: (i, 0)),
        ],
        out_shape=[_f32((N, H)), _f32((2, N, H)), _f32((N, H))],
    )(hh, aW, ab, bW, bb, dW, db_, eW, eb)


def _edge_lin1_body(e, wf, bf, out):
    ce = jnp.dot(1.0 / e[...], wf[...], preferred_element_type=F32) + bf[...]
    out[0] = ce[:, :HH]
    out[1] = ce[:, HH:]


def _edge_linear1(e, wf, bf):
    return pl.pallas_call(
        _edge_lin1_body,
        grid=(E // EBLK,),
        in_specs=[
            pl.BlockSpec((EBLK, 16), lambda i: (i, 0)),
            pl.BlockSpec((16, H), lambda i: (0, 0)),
            pl.BlockSpec((1, H), lambda i: (0, 0)),
        ],
        out_specs=pl.BlockSpec((2, EBLK, HH), lambda i: (0, i, 0)),
        out_shape=_f32((2, E, HH)),
    )(e, wf, bf)


def _edge_lin2_body(e, enew, stats, wf, bf, cW, g, b, out):
    st = stats[:, 0, :]
    s1h0 = jnp.sum(st[:NUM_TILES, :HH], axis=0, keepdims=True)
    s2h0 = jnp.sum(st[:NUM_TILES, HH:], axis=0, keepdims=True)
    s1h1 = jnp.sum(st[NUM_TILES:, :HH], axis=0, keepdims=True)
    s2h1 = jnp.sum(st[NUM_TILES:, HH:], axis=0, keepdims=True)
    inv_e = 1.0 / float(E)
    m0, m1 = s1h0 * inv_e, s1h1 * inv_e
    v0 = s2h0 * inv_e - m0 * m0
    v1 = s2h1 * inv_e - m1 * m1
    r0 = (enew[0] - m0) * lax.rsqrt(v0 + 1e-5) * g[:, :HH] + b[:, :HH]
    r1 = (enew[1] - m1) * lax.rsqrt(v1 + 1e-5) * g[:, HH:] + b[:, HH:]
    r0 = jnp.maximum(r0, 0.0)
    r1 = jnp.maximum(r1, 0.0)
    ce = (jnp.dot(1.0 / e[...], wf[...], preferred_element_type=F32) + bf[...]
          + jnp.dot(r0, cW[:HH, :], preferred_element_type=F32)
          + jnp.dot(r1, cW[HH:, :], preferred_element_type=F32))
    out[0] = ce[:, :HH]
    out[1] = ce[:, HH:]


def _edge_linear2(e, enew, stats, wf, bf, cW, g, b):
    return pl.pallas_call(
        _edge_lin2_body,
        grid=(E // EBLK,),
        in_specs=[
            pl.BlockSpec((EBLK, 16), lambda i: (i, 0)),
            pl.BlockSpec((2, EBLK, HH), lambda i: (0, i, 0)),
            pl.BlockSpec((2 * NUM_TILES, 8, H), lambda i: (0, 0, 0)),
            pl.BlockSpec((16, H), lambda i: (0, 0)),
            pl.BlockSpec((1, H), lambda i: (0, 0)),
            pl.BlockSpec((H, H), lambda i: (0, 0)),
            pl.BlockSpec((1, H), lambda i: (0, 0)),
            pl.BlockSpec((1, H), lambda i: (0, 0)),
        ],
        out_specs=pl.BlockSpec((2, EBLK, HH), lambda i: (0, i, 0)),
        out_shape=_f32((2, E, HH)),
    )(e, enew, stats, wf, bf, cW, g, b)


def _node_upd_a_body(ah, acc, raw, ps, ps2):
    num = jnp.concatenate([acc[0, :, :HH], acc[1, :, :HH]], axis=1)
    den = jnp.concatenate([acc[0, :, HH:], acc[1, :, HH:]], axis=1)
    r = ah[...] + num / (den + 1e-6)
    raw[...] = r
    ps[...] = jnp.sum(r, axis=0, keepdims=True).reshape(1, 1, H)
    ps2[...] = jnp.sum(r * r, axis=0, keepdims=True).reshape(1, 1, H)


def _node_update_a(ah, acc):
    return pl.pallas_call(
        _node_upd_a_body,
        grid=(N // NBLK,),
        in_specs=[
            pl.BlockSpec((NBLK, H), lambda i: (i, 0)),
            pl.BlockSpec((2, NBLK, H), lambda i: (0, i, 0)),
        ],
        out_specs=[
            pl.BlockSpec((NBLK, H), lambda i: (i, 0)),
            pl.BlockSpec((1, 1, H), lambda i: (i, 0, 0)),
            pl.BlockSpec((1, 1, H), lambda i: (i, 0, 0)),
        ],
        out_shape=[_f32((N, H)), _f32((N // NBLK, 1, H)), _f32((N // NBLK, 1, H))],
    )(ah, acc)


def _node_upd_b_body(raw, ps, ps2, g, b, hin, out, psh):
    inv_n = 1.0 / float(N)
    m = jnp.sum(ps[...], axis=0) * inv_n
    v = jnp.sum(ps2[...], axis=0) * inv_n - m * m
    hn = (raw[...] - m) * lax.rsqrt(v + 1e-5) * g[...] + b[...]
    r = hin[...] + jnp.maximum(hn, 0.0)
    out[...] = r
    psh[...] = jnp.sum(r, axis=0, keepdims=True).reshape(1, 1, H)


def _node_update_b(raw, ps, ps2, g, b, hin):
    return pl.pallas_call(
        _node_upd_b_body,
        grid=(N // NBLK,),
        in_specs=[
            pl.BlockSpec((NBLK, H), lambda i: (i, 0)),
            pl.BlockSpec((N // NBLK, 1, H), lambda i: (0, 0, 0)),
            pl.BlockSpec((N // NBLK, 1, H), lambda i: (0, 0, 0)),
            pl.BlockSpec((1, H), lambda i: (0, 0)),
            pl.BlockSpec((1, H), lambda i: (0, 0)),
            pl.BlockSpec((NBLK, H), lambda i: (i, 0)),
        ],
        out_specs=[
            pl.BlockSpec((NBLK, H), lambda i: (i, 0)),
            pl.BlockSpec((1, 1, H), lambda i: (i, 0, 0)),
        ],
        out_shape=[_f32((N, H)), _f32((N // NBLK, 1, H))],
    )(raw, ps, ps2, g, b, hin)


def _head_body(psh, st, w1, b1, w2, b2, w3, b3, out):
    hm = jnp.sum(psh[...], axis=0) * (1.0 / float(N))
    z = jnp.concatenate([hm, st[...]], axis=1)
    z = jnp.maximum(jnp.dot(z, w1[...], preferred_element_type=F32) + b1[...], 0.0)
    z = jnp.maximum(jnp.dot(z, w2[...], preferred_element_type=F32) + b2[...], 0.0)
    out[...] = jnp.tanh(jnp.dot(z, w3[...], preferred_element_type=F32) + b3[...])


def _head(psh, st, w1, b1, w2, b2, w3, b3):
    return pl.pallas_call(_head_body, out_shape=_f32((1, 2)))(
        psh, st, w1, b1, w2, b2, w3, b3)


def _sigmoid16(x):
    return 1.0 / (1.0 + jnp.exp(-x))


def _sc_body(do_enew, *refs):
    if do_enew:
        (ce, dbt, eht, src, dst, accout, enew, bnstats,
         src_v0, src_v1, dst_v0, dst_v1, dsts_v0, dsts_v1,
         ce_v0, ce_v1, db_v0, db_v1, eh_v0, eh_v1,
         stat_v,
         s_src0, s_src1, s_dst0, s_dst1, s_ce0, s_ce1,
         s_gdb0, s_gdb1, s_geh0, s_geh1, s_scat, s_en0, s_en1,
         acc_sh) = refs
        s_en = [s_en0, s_en1]
    else:
        (ce, dbt, eht, src, dst, accout,
         src_v0, src_v1, dst_v0, dst_v1, dsts_v0, dsts_v1,
         ce_v0, ce_v1, db_v0, db_v1, eh_v0, eh_v1,
         stat_v,
         s_src0, s_src1, s_dst0, s_dst1, s_ce0, s_ce1,
         s_gdb0, s_gdb1, s_geh0, s_geh1, s_scat,
         acc_sh) = refs
    src_vs = [src_v0, src_v1]
    dst_vs = [dst_v0, dst_v1]
    dsts_vs = [dsts_v0, dsts_v1]
    ce_vs = [ce_v0, ce_v1]
    db_vs = [db_v0, db_v1]
    eh_vs = [eh_v0, eh_v1]
    s_src = [s_src0, s_src1]
    s_dst = [s_dst0, s_dst1]
    s_ce = [s_ce0, s_ce1]
    s_gdb = [s_gdb0, s_gdb1]
    s_geh = [s_geh0, s_geh1]

    cid = lax.axis_index("c")
    sid = lax.axis_index("s")

    def zfill(i, _):
        for c in range(H // 16):
            db_v0[i, pl.ds(c * 16, 16)] = jnp.zeros((16,), F32)
        return 0
    lax.fori_loop(0, ZROWS, zfill, 0)
    base = sid * DRAIN_ROWS
    zsrc = db_v0.at[pl.ds(0, ZROWS)]

    def zcopy(k, _):
        pltpu.sync_copy(zsrc, acc_sh.at[pl.ds(base + k * ZROWS, ZROWS)])
        return 0
    lax.fori_loop(0, DRAIN_ROWS // ZROWS, zcopy, 0)

    @pl.when(sid == NUM_TILES - 1)
    def _():
        pltpu.sync_copy(zsrc, acc_sh.at[pl.ds(N - ZROWS, ZROWS)])
    plsc.subcore_barrier()

    nblk = (NUM_EBLK - sid + NUM_TILES - 1) // NUM_TILES
    tbl_off = cid * N

    def _off(j):
        return (sid + j * NUM_TILES) * B

    def idx_issue(slot, j):
        o = _off(j)
        pltpu.async_copy(src.at[pl.ds(o, B)], src_vs[slot], s_src[slot])
        pltpu.async_copy(dst.at[pl.ds(o, B)], dst_vs[slot], s_dst[slot])

    def idx_wait(slot):
        pltpu.make_async_copy(src.at[pl.ds(0, B)], src_vs[slot], s_src[slot]).wait()
        pltpu.make_async_copy(dst.at[pl.ds(0, B)], dst_vs[slot], s_dst[slot]).wait()

    def ce_issue(slot, j):
        pltpu.async_copy(ce.at[pl.ds(cid * E + _off(j), B)], ce_vs[slot],
                         s_ce[slot])

    def ce_wait(slot):
        pltpu.make_async_copy(ce.at[pl.ds(0, B)], ce_vs[slot], s_ce[slot]).wait()

    def adjust(slot):
        for c in range(B // 16):
            cs = pl.ds(c * 16, 16)
            src_vs[slot][cs] = src_vs[slot][cs] + tbl_off
            dsts_vs[slot][cs] = dst_vs[slot][cs]

    def gather_issue(slot):
        pltpu.async_copy(dbt.at[src_vs[slot]], db_vs[slot], s_gdb[slot])
        pltpu.async_copy(eht.at[dsts_vs[slot]], eh_vs[slot], s_geh[slot])

    def gather_wait(slot):
        pltpu.make_async_copy(dbt.at[src_vs[slot]], db_vs[slot],
                              s_gdb[slot]).wait()
        pltpu.make_async_copy(eht.at[dsts_vs[slot]], eh_vs[slot],
                              s_geh[slot]).wait()

    def scat_wait():
        pltpu.make_async_copy(db_v0, acc_sh.at[dsts_v0], s_scat).wait()

    def enew_wait(slot):
        pltpu.make_async_copy(ce_vs[slot], enew.at[pl.ds(0, B)], s_en[slot]).wait()

    def block(j, slot, stats):
        oth = 1 - slot

        @pl.when(j + 1 < nblk)
        def _():
            idx_wait(oth)

        @pl.when(j >= 1)
        def _():
            scat_wait()
        if do_enew:
            @pl.when(j >= 1)
            def _():
                enew_wait(oth)

        @pl.when(j + 1 < nblk)
        def _():
            ce_issue(oth, j + 1)
            adjust(oth)
            gather_issue(oth)

        gather_wait(slot)
        ce_wait(slot)

        @pl.when(j + 2 < nblk)
        def _():
            idx_issue(slot, j + 2)

        def edge(e_i, st):
            st = list(st)
            for c in range(HH // 16):
                ds = pl.ds(c * 16, 16)
                dsb = pl.ds(HH + c * 16, 16)
                en = (ce_vs[slot][e_i, ds] + db_vs[slot][e_i, ds]
                      + eh_vs[slot][e_i, pl.ds(cid * HH + c * 16, 16)])
                if do_enew:
                    ce_vs[slot][e_i, ds] = en
                    st[c] = st[c] + en
                    st[4 + c] = st[4 + c] + en * en
                sg = _sigmoid16(en)
                num = sg * db_vs[slot][e_i, dsb]
                db_vs[slot][e_i, ds] = num
                db_vs[slot][e_i, dsb] = sg
            return tuple(st)

        stats = plsc.parallel_loop(0, B, 1, unroll=4, carry=tuple(stats))(edge)
        if do_enew:
            pltpu.async_copy(ce_vs[slot], enew.at[pl.ds(cid * E + _off(j), B)],
                             s_en[slot])
        pltpu.async_copy(db_vs[slot], acc_sh.at[dsts_vs[slot]], s_scat, add=True)
        return stats

    idx_issue(0, 0)
    idx_wait(0)
    ce_issue(0, 0)
    adjust(0)
    gather_issue(0)
    idx_issue(1, 1)

    zstats = tuple(jnp.zeros((16,), F32) for _ in range(8))
    npair = nblk // 2

    def pair(g, stats):
        stats = block(2 * g, 0, stats)
        return block(2 * g + 1, 1, stats)

    stats = lax.fori_loop(0, npair, pair, zstats)
    stats = lax.fori_loop(2 * npair, nblk, lambda j, st: block(j, 0, st), stats)

    scat_wait()
    if do_enew:
        @pl.when(nblk % 2 == 1)
        def _():
            enew_wait(0)

        @pl.when(nblk % 2 == 0)
        def _():
            enew_wait(1)
        for c in range(8):
            stat_v[0, 0, pl.ds(c * 16, 16)] = stats[c]
        pltpu.sync_copy(stat_v, bnstats.at[pl.ds(cid * NUM_TILES + sid, 1)])

    plsc.subcore_barrier()
    pltpu.sync_copy(
        acc_sh.at[pl.ds(base, DRAIN_ROWS)],
        accout.at[pl.ds(cid * N + base, DRAIN_ROWS)])

    @pl.when(sid == NUM_TILES - 1)
    def _():
        pltpu.sync_copy(
            acc_sh.at[pl.ds(N - ZROWS, ZROWS)],
            accout.at[pl.ds(cid * N + N - ZROWS, ZROWS)])


def _make_sc_edge_pass(do_enew):
    out_type = [_f32((2 * N, H))]
    if do_enew:
        out_type += [_f32((2 * E, HH)), _f32((2 * NUM_TILES, 8, H))]
    scratch = [
        pltpu.VMEM((B,), jnp.int32),
        pltpu.VMEM((B,), jnp.int32),
        pltpu.VMEM((B,), jnp.int32),
        pltpu.VMEM((B,), jnp.int32),
        pltpu.VMEM((B,), jnp.int32),
        pltpu.VMEM((B,), jnp.int32),
        pltpu.VMEM((B, HH), F32),
        pltpu.VMEM((B, HH), F32),
        pltpu.VMEM((B, H), F32),
        pltpu.VMEM((B, H), F32),
        pltpu.VMEM((B, H), F32),
        pltpu.VMEM((B, H), F32),
        pltpu.VMEM((1, 8, H), F32),
        pltpu.SemaphoreType.DMA,
        pltpu.SemaphoreType.DMA,
        pltpu.SemaphoreType.DMA,
        pltpu.SemaphoreType.DMA,
        pltpu.SemaphoreType.DMA,
        pltpu.SemaphoreType.DMA,
        pltpu.SemaphoreType.DMA,
        pltpu.SemaphoreType.DMA,
        pltpu.SemaphoreType.DMA,
        pltpu.SemaphoreType.DMA,
        pltpu.SemaphoreType.DMA,
    ]
    if do_enew:
        scratch += [pltpu.SemaphoreType.DMA, pltpu.SemaphoreType.DMA]
    scratch += [pltpu.VMEM_SHARED((N, H), F32)]
    return pl.kernel(
        functools.partial(_sc_body, do_enew),
        out_type=out_type,
        mesh=plsc.VectorSubcoreMesh(core_axis_name="c", subcore_axis_name="s"),
        scratch_types=scratch,
    )


_sc_pass1 = _make_sc_edge_pass(True)
_sc_pass2 = _make_sc_edge_pass(False)


def kernel(h, e, state, params, edge_index):
    p = params
    l1, l2 = p["layers"]
    r2 = lambda b: b.reshape(1, H)
    src = edge_index[0]
    dst = edge_index[1]

    wf1, bf1, wf2, bf2 = _fold_weights(
        p["emb_e_W"], p["emb_e_b"].reshape(1, H),
        l1["C_W"], r2(l1["C_b"]), l2["C_W"], r2(l2["C_b"]))

    hh = _embed(h, p["emb_h_W"], p["emb_h_b"].reshape(1, H))

    ah, dbt, eht = _node_matmuls(
        hh, l1["A_W"], r2(l1["A_b"]), l1["B_W"], r2(l1["B_b"]),
        l1["D_W"], r2(l1["D_b"]), l1["E_W"], r2(l1["E_b"]))
    ce1 = _edge_linear1(e, wf1, bf1)
    acc1, enew1, stats1 = _sc_pass1(
        ce1.reshape(2 * E, HH), dbt.reshape(2 * N, H), eht, src, dst)
    raw1, ps1, ps21 = _node_update_a(ah, acc1.reshape(2, N, H))
    hh1, _ = _node_update_b(raw1, ps1, ps21, r2(l1["bn_h_g"]),
                            r2(l1["bn_h_b"]), hh)

    ah2, dbt2, eht2 = _node_matmuls(
        hh1, l2["A_W"], r2(l2["A_b"]), l2["B_W"], r2(l2["B_b"]),
        l2["D_W"], r2(l2["D_b"]), l2["E_W"], r2(l2["E_b"]))
    ce2 = _edge_linear2(e, enew1.reshape(2, E, HH), stats1, wf2, bf2,
                        l2["C_W"], r2(l1["bn_e_g"]), r2(l1["bn_e_b"]))
    (acc2,) = _sc_pass2(
        ce2.reshape(2 * E, HH), dbt2.reshape(2 * N, H), eht2, src, dst)
    raw2, ps2, ps22 = _node_update_a(ah2, acc2.reshape(2, N, H))
    hh2, psh = _node_update_b(raw2, ps2, ps22, r2(l2["bn_h_g"]),
                              r2(l2["bn_h_b"]), hh1)

    return _head(psh, state, p["l1_W"], p["l1_b"].reshape(1, 256),
                 p["l2_W"], p["l2_b"].reshape(1, 256),
                 p["l3_W"], p["l3_b"].reshape(1, 2))


# speedup vs baseline: 2.7950x; 1.5905x over previous
import functools

import jax
import jax.numpy as jnp
from jax import lax
from jax.experimental import pallas as pl
from jax.experimental.pallas import tpu as pltpu
from jax.experimental.pallas import tpu_sc as plsc

N = 10000
E = 320000
H = 128
HH = 64
NBLK = 1000
EBLK = 2000
B = 64
NUM_TILES = 16
DRAIN_ROWS = 624
ZROWS = 16
NUM_EBLK = E // B
F32 = jnp.float32


def _f32(x):
    return jax.ShapeDtypeStruct(x, F32)


def _fold_body(embW, embb, c1W, c1b, c2W, c2b, wf1, bf1, wf2, bf2):
    wf1[...] = jnp.dot(embW[...], c1W[...], preferred_element_type=F32)
    bf1[...] = jnp.dot(embb[...], c1W[...], preferred_element_type=F32) + c1b[...]
    wf2[...] = jnp.dot(embW[...], c2W[...], preferred_element_type=F32)
    bf2[...] = jnp.dot(embb[...], c2W[...], preferred_element_type=F32) + c2b[...]


def _fold_weights(embW, embb, c1W, c1b, c2W, c2b):
    return pl.pallas_call(
        _fold_body,
        out_shape=[_f32((16, H)), _f32((1, H)), _f32((16, H)), _f32((1, H))],
    )(embW, embb, c1W, c1b, c2W, c2b)


def _embed_body(h, w, b, out):
    out[...] = jnp.dot(h[...], w[...], preferred_element_type=F32) + b[...]


def _embed(h, w, b):
    return pl.pallas_call(
        _embed_body,
        grid=(N // NBLK,),
        in_specs=[
            pl.BlockSpec((NBLK, H), lambda i: (i, 0)),
            pl.BlockSpec((H, H), lambda i: (0, 0)),
            pl.BlockSpec((1, H), lambda i: (0, 0)),
        ],
        out_specs=pl.BlockSpec((NBLK, H), lambda i: (i, 0)),
        out_shape=_f32((N, H)),
    )(h, w, b)


def _node_mm_body(hh, aW, ab, bW, bb, dW, db_, eW, eb, ah, dbt, eht):
    x = hh[...]
    ah[...] = jnp.dot(x, aW[...], preferred_element_type=F32) + ab[...]
    Bh = jnp.dot(x, bW[...], preferred_element_type=F32) + bb[...]
    Dh = jnp.dot(x, dW[...], preferred_element_type=F32) + db_[...]
    Eh = jnp.dot(x, eW[...], preferred_element_type=F32) + eb[...]
    dbt[0] = jnp.concatenate([Dh[:, :HH], Bh[:, :HH]], axis=1)
    dbt[1] = jnp.concatenate([Dh[:, HH:], Bh[:, HH:]], axis=1)
    eht[...] = Eh


def _node_matmuls(hh, aW, ab, bW, bb, dW, db_, eW, eb):
    return pl.pallas_call(
        _node_mm_body,
        grid=(N // NBLK,),
        in_specs=[pl.BlockSpec((NBLK, H), lambda i: (i, 0))]
        + [pl.BlockSpec((H, H), lambda i: (0, 0)),
           pl.BlockSpec((1, H), lambda i: (0, 0))] * 4,
        out_specs=[
            pl.BlockSpec((NBLK, H), lambda i: (i, 0)),
            pl.BlockSpec((2, NBLK, H), lambda i: (0, i, 0)),
            pl.BlockSpec((NBLK, H), lambda i: (i, 0)),
        ],
        out_shape=[_f32((N, H)), _f32((2, N, H)), _f32((N, H))],
    )(hh, aW, ab, bW, bb, dW, db_, eW, eb)


def _edge_lin1_body(e, wf, bf, out):
    ce = jnp.dot(1.0 / e[...], wf[...], preferred_element_type=F32) + bf[...]
    out[0] = ce[:, :HH]
    out[1] = ce[:, HH:]


def _edge_linear1(e, wf, bf):
    return pl.pallas_call(
        _edge_lin1_body,
        grid=(E // EBLK,),
        in_specs=[
            pl.BlockSpec((EBLK, 16), lambda i: (i, 0)),
            pl.BlockSpec((16, H), lambda i: (0, 0)),
            pl.BlockSpec((1, H), lambda i: (0, 0)),
        ],
        out_specs=pl.BlockSpec((2, EBLK, HH), lambda i: (0, i, 0)),
        out_shape=_f32((2, E, HH)),
    )(e, wf, bf)


def _edge_stats_body(enew, ps, ps2):
    x0 = enew[0]
    x1 = enew[1]
    s1 = jnp.concatenate([jnp.sum(x0, axis=0, keepdims=True),
                          jnp.sum(x1, axis=0, keepdims=True)], axis=1)
    s2 = jnp.concatenate([jnp.sum(x0 * x0, axis=0, keepdims=True),
                          jnp.sum(x1 * x1, axis=0, keepdims=True)], axis=1)
    ps[...] = s1.reshape(1, 1, H)
    ps2[...] = s2.reshape(1, 1, H)


def _edge_stats(enew):
    return pl.pallas_call(
        _edge_stats_body,
        grid=(E // EBLK,),
        in_specs=[pl.BlockSpec((2, EBLK, HH), lambda i: (0, i, 0))],
        out_specs=[
            pl.BlockSpec((1, 1, H), lambda i: (i, 0, 0)),
            pl.BlockSpec((1, 1, H), lambda i: (i, 0, 0)),
        ],
        out_shape=[_f32((E // EBLK, 1, H)), _f32((E // EBLK, 1, H))],
    )(enew)


def _edge_lin2_body(e, enew, ps, ps2, wf, bf, cW, g, b, out):
    inv_e = 1.0 / float(E)
    s1 = jnp.sum(ps[...], axis=0)
    s2 = jnp.sum(ps2[...], axis=0)
    m0, m1 = s1[:, :HH] * inv_e, s1[:, HH:] * inv_e
    s2h0, s2h1 = s2[:, :HH], s2[:, HH:]
    v0 = s2h0 * inv_e - m0 * m0
    v1 = s2h1 * inv_e - m1 * m1
    r0 = (enew[0] - m0) * lax.rsqrt(v0 + 1e-5) * g[:, :HH] + b[:, :HH]
    r1 = (enew[1] - m1) * lax.rsqrt(v1 + 1e-5) * g[:, HH:] + b[:, HH:]
    r0 = jnp.maximum(r0, 0.0)
    r1 = jnp.maximum(r1, 0.0)
    ce = (jnp.dot(1.0 / e[...], wf[...], preferred_element_type=F32) + bf[...]
          + jnp.dot(r0, cW[:HH, :], preferred_element_type=F32)
          + jnp.dot(r1, cW[HH:, :], preferred_element_type=F32))
    out[0] = ce[:, :HH]
    out[1] = ce[:, HH:]


def _edge_linear2(e, enew, ps, ps2, wf, bf, cW, g, b):
    return pl.pallas_call(
        _edge_lin2_body,
        grid=(E // EBLK,),
        in_specs=[
            pl.BlockSpec((EBLK, 16), lambda i: (i, 0)),
            pl.BlockSpec((2, EBLK, HH), lambda i: (0, i, 0)),
            pl.BlockSpec((E // EBLK, 1, H), lambda i: (0, 0, 0)),
            pl.BlockSpec((E // EBLK, 1, H), lambda i: (0, 0, 0)),
            pl.BlockSpec((16, H), lambda i: (0, 0)),
            pl.BlockSpec((1, H), lambda i: (0, 0)),
            pl.BlockSpec((H, H), lambda i: (0, 0)),
            pl.BlockSpec((1, H), lambda i: (0, 0)),
            pl.BlockSpec((1, H), lambda i: (0, 0)),
        ],
        out_specs=pl.BlockSpec((2, EBLK, HH), lambda i: (0, i, 0)),
        out_shape=_f32((2, E, HH)),
    )(e, enew, ps, ps2, wf, bf, cW, g, b)


def _node_upd_a_body(ah, acc, raw, ps, ps2):
    num = jnp.concatenate([acc[0, :, :HH], acc[1, :, :HH]], axis=1)
    den = jnp.concatenate([acc[0, :, HH:], acc[1, :, HH:]], axis=1)
    r = ah[...] + num / (den + 1e-6)
    raw[...] = r
    ps[...] = jnp.sum(r, axis=0, keepdims=True).reshape(1, 1, H)
    ps2[...] = jnp.sum(r * r, axis=0, keepdims=True).reshape(1, 1, H)


def _node_update_a(ah, acc):
    return pl.pallas_call(
        _node_upd_a_body,
        grid=(N // NBLK,),
        in_specs=[
            pl.BlockSpec((NBLK, H), lambda i: (i, 0)),
            pl.BlockSpec((2, NBLK, H), lambda i: (0, i, 0)),
        ],
        out_specs=[
            pl.BlockSpec((NBLK, H), lambda i: (i, 0)),
            pl.BlockSpec((1, 1, H), lambda i: (i, 0, 0)),
            pl.BlockSpec((1, 1, H), lambda i: (i, 0, 0)),
        ],
        out_shape=[_f32((N, H)), _f32((N // NBLK, 1, H)), _f32((N // NBLK, 1, H))],
    )(ah, acc)


def _node_upd_b_body(raw, ps, ps2, g, b, hin, out, psh):
    inv_n = 1.0 / float(N)
    m = jnp.sum(ps[...], axis=0) * inv_n
    v = jnp.sum(ps2[...], axis=0) * inv_n - m * m
    hn = (raw[...] - m) * lax.rsqrt(v + 1e-5) * g[...] + b[...]
    r = hin[...] + jnp.maximum(hn, 0.0)
    out[...] = r
    psh[...] = jnp.sum(r, axis=0, keepdims=True).reshape(1, 1, H)


def _node_update_b(raw, ps, ps2, g, b, hin):
    return pl.pallas_call(
        _node_upd_b_body,
        grid=(N // NBLK,),
        in_specs=[
            pl.BlockSpec((NBLK, H), lambda i: (i, 0)),
            pl.BlockSpec((N // NBLK, 1, H), lambda i: (0, 0, 0)),
            pl.BlockSpec((N // NBLK, 1, H), lambda i: (0, 0, 0)),
            pl.BlockSpec((1, H), lambda i: (0, 0)),
            pl.BlockSpec((1, H), lambda i: (0, 0)),
            pl.BlockSpec((NBLK, H), lambda i: (i, 0)),
        ],
        out_specs=[
            pl.BlockSpec((NBLK, H), lambda i: (i, 0)),
            pl.BlockSpec((1, 1, H), lambda i: (i, 0, 0)),
        ],
        out_shape=[_f32((N, H)), _f32((N // NBLK, 1, H))],
    )(raw, ps, ps2, g, b, hin)


def _head_body(psh, st, w1, b1, w2, b2, w3, b3, out):
    hm = jnp.sum(psh[...], axis=0) * (1.0 / float(N))
    z = jnp.concatenate([hm, st[...]], axis=1)
    z = jnp.maximum(jnp.dot(z, w1[...], preferred_element_type=F32) + b1[...], 0.0)
    z = jnp.maximum(jnp.dot(z, w2[...], preferred_element_type=F32) + b2[...], 0.0)
    out[...] = jnp.tanh(jnp.dot(z, w3[...], preferred_element_type=F32) + b3[...])


def _head(psh, st, w1, b1, w2, b2, w3, b3):
    return pl.pallas_call(_head_body, out_shape=_f32((1, 2)))(
        psh, st, w1, b1, w2, b2, w3, b3)


def _sigmoid16(x):
    return 1.0 / (1.0 + jnp.exp(-x))


def _sc_body(do_enew, *refs):
    if do_enew:
        (ce, dbt, eht, src, dst, accout, enew,
         src_v0, src_v1, dst_v0, dst_v1, dsts_v0, dsts_v1,
         ce_v0, ce_v1, db_v0, db_v1, eh_v0, eh_v1,
         s_src0, s_src1, s_dst0, s_dst1, s_ce0, s_ce1,
         s_gdb0, s_gdb1, s_geh0, s_geh1, s_scat, s_en0, s_en1,
         acc_sh) = refs
        s_en = [s_en0, s_en1]
    else:
        (ce, dbt, eht, src, dst, accout,
         src_v0, src_v1, dst_v0, dst_v1, dsts_v0, dsts_v1,
         ce_v0, ce_v1, db_v0, db_v1, eh_v0, eh_v1,
         s_src0, s_src1, s_dst0, s_dst1, s_ce0, s_ce1,
         s_gdb0, s_gdb1, s_geh0, s_geh1, s_scat,
         acc_sh) = refs
    src_vs = [src_v0, src_v1]
    dst_vs = [dst_v0, dst_v1]
    dsts_vs = [dsts_v0, dsts_v1]
    ce_vs = [ce_v0, ce_v1]
    db_vs = [db_v0, db_v1]
    eh_vs = [eh_v0, eh_v1]
    s_src = [s_src0, s_src1]
    s_dst = [s_dst0, s_dst1]
    s_ce = [s_ce0, s_ce1]
    s_gdb = [s_gdb0, s_gdb1]
    s_geh = [s_geh0, s_geh1]

    cid = lax.axis_index("c")
    sid = lax.axis_index("s")

    def zfill(i, _):
        for c in range(H // 16):
            db_v0[i, pl.ds(c * 16, 16)] = jnp.zeros((16,), F32)
        return 0
    lax.fori_loop(0, ZROWS, zfill, 0)
    base = sid * DRAIN_ROWS
    zsrc = db_v0.at[pl.ds(0, ZROWS)]

    def zcopy(k, _):
        pltpu.sync_copy(zsrc, acc_sh.at[pl.ds(base + k * ZROWS, ZROWS)])
        return 0
    lax.fori_loop(0, DRAIN_ROWS // ZROWS, zcopy, 0)

    @pl.when(sid == NUM_TILES - 1)
    def _():
        pltpu.sync_copy(zsrc, acc_sh.at[pl.ds(N - ZROWS, ZROWS)])
    plsc.subcore_barrier()

    nblk = (NUM_EBLK - sid + NUM_TILES - 1) // NUM_TILES
    tbl_off = cid * N

    def _off(j):
        return (sid + j * NUM_TILES) * B

    def idx_issue(slot, j):
        o = _off(j)
        pltpu.async_copy(src.at[pl.ds(o, B)], src_vs[slot], s_src[slot])
        pltpu.async_copy(dst.at[pl.ds(o, B)], dst_vs[slot], s_dst[slot])

    def idx_wait(slot):
        pltpu.make_async_copy(src.at[pl.ds(0, B)], src_vs[slot], s_src[slot]).wait()
        pltpu.make_async_copy(dst.at[pl.ds(0, B)], dst_vs[slot], s_dst[slot]).wait()

    def ce_issue(slot, j):
        pltpu.async_copy(ce.at[pl.ds(cid * E + _off(j), B)], ce_vs[slot],
                         s_ce[slot])

    def ce_wait(slot):
        pltpu.make_async_copy(ce.at[pl.ds(0, B)], ce_vs[slot], s_ce[slot]).wait()

    def adjust(slot):
        for c in range(B // 16):
            cs = pl.ds(c * 16, 16)
            src_vs[slot][cs] = src_vs[slot][cs] + tbl_off
            dsts_vs[slot][cs] = dst_vs[slot][cs]

    def gather_issue(slot):
        pltpu.async_copy(dbt.at[src_vs[slot]], db_vs[slot], s_gdb[slot])
        pltpu.async_copy(eht.at[dsts_vs[slot]], eh_vs[slot], s_geh[slot])

    def gather_wait(slot):
        pltpu.make_async_copy(dbt.at[src_vs[slot]], db_vs[slot],
                              s_gdb[slot]).wait()
        pltpu.make_async_copy(eht.at[dsts_vs[slot]], eh_vs[slot],
                              s_geh[slot]).wait()

    def scat_wait():
        pltpu.make_async_copy(db_v0, acc_sh.at[dsts_v0], s_scat).wait()

    def enew_wait(slot):
        pltpu.make_async_copy(ce_vs[slot], enew.at[pl.ds(0, B)], s_en[slot]).wait()

    def block(j, slot, _carry):
        oth = 1 - slot

        @pl.when(j + 1 < nblk)
        def _():
            idx_wait(oth)

        @pl.when(j >= 1)
        def _():
            scat_wait()
        if do_enew:
            @pl.when(j >= 1)
            def _():
                enew_wait(oth)

        @pl.when(j + 1 < nblk)
        def _():
            ce_issue(oth, j + 1)
            adjust(oth)
            gather_issue(oth)

        gather_wait(slot)
        ce_wait(slot)

        @pl.when(j + 2 < nblk)
        def _():
            idx_issue(slot, j + 2)

        @plsc.parallel_loop(0, B, 1, unroll=4)
        def edge(e_i):
            for c in range(HH // 16):
                ds = pl.ds(c * 16, 16)
                dsb = pl.ds(HH + c * 16, 16)
                en = (ce_vs[slot][e_i, ds] + db_vs[slot][e_i, ds]
                      + eh_vs[slot][e_i, pl.ds(cid * HH + c * 16, 16)])
                if do_enew:
                    ce_vs[slot][e_i, ds] = en
                sg = _sigmoid16(en)
                num = sg * db_vs[slot][e_i, dsb]
                db_vs[slot][e_i, ds] = num
                db_vs[slot][e_i, dsb] = sg

        if do_enew:
            pltpu.async_copy(ce_vs[slot], enew.at[pl.ds(cid * E + _off(j), B)],
                             s_en[slot])
        pltpu.async_copy(db_vs[slot], acc_sh.at[dsts_vs[slot]], s_scat, add=True)
        return 0

    idx_issue(0, 0)
    idx_wait(0)
    ce_issue(0, 0)
    adjust(0)
    gather_issue(0)
    idx_issue(1, 1)

    npair = nblk // 2

    def pair(g, carry):
        block(2 * g, 0, carry)
        return block(2 * g + 1, 1, carry)

    lax.fori_loop(0, npair, pair, 0)
    lax.fori_loop(2 * npair, nblk, lambda j, c_: block(j, 0, c_), 0)

    scat_wait()
    if do_enew:
        @pl.when(nblk % 2 == 1)
        def _():
            enew_wait(0)

        @pl.when(nblk % 2 == 0)
        def _():
            enew_wait(1)

    plsc.subcore_barrier()
    pltpu.sync_copy(
        acc_sh.at[pl.ds(base, DRAIN_ROWS)],
        accout.at[pl.ds(cid * N + base, DRAIN_ROWS)])

    @pl.when(sid == NUM_TILES - 1)
    def _():
        pltpu.sync_copy(
            acc_sh.at[pl.ds(N - ZROWS, ZROWS)],
            accout.at[pl.ds(cid * N + N - ZROWS, ZROWS)])


def _make_sc_edge_pass(do_enew):
    out_type = [_f32((2 * N, H))]
    if do_enew:
        out_type += [_f32((2 * E, HH))]
    scratch = [
        pltpu.VMEM((B,), jnp.int32),
        pltpu.VMEM((B,), jnp.int32),
        pltpu.VMEM((B,), jnp.int32),
        pltpu.VMEM((B,), jnp.int32),
        pltpu.VMEM((B,), jnp.int32),
        pltpu.VMEM((B,), jnp.int32),
        pltpu.VMEM((B, HH), F32),
        pltpu.VMEM((B, HH), F32),
        pltpu.VMEM((B, H), F32),
        pltpu.VMEM((B, H), F32),
        pltpu.VMEM((B, H), F32),
        pltpu.VMEM((B, H), F32),
        pltpu.SemaphoreType.DMA,
        pltpu.SemaphoreType.DMA,
        pltpu.SemaphoreType.DMA,
        pltpu.SemaphoreType.DMA,
        pltpu.SemaphoreType.DMA,
        pltpu.SemaphoreType.DMA,
        pltpu.SemaphoreType.DMA,
        pltpu.SemaphoreType.DMA,
        pltpu.SemaphoreType.DMA,
        pltpu.SemaphoreType.DMA,
        pltpu.SemaphoreType.DMA,
    ]
    if do_enew:
        scratch += [pltpu.SemaphoreType.DMA, pltpu.SemaphoreType.DMA]
    scratch += [pltpu.VMEM_SHARED((N, H), F32)]
    return pl.kernel(
        functools.partial(_sc_body, do_enew),
        out_type=out_type,
        mesh=plsc.VectorSubcoreMesh(core_axis_name="c", subcore_axis_name="s"),
        scratch_types=scratch,
    )


_sc_pass1 = _make_sc_edge_pass(True)
_sc_pass2 = _make_sc_edge_pass(False)


def kernel(h, e, state, params, edge_index):
    p = params
    l1, l2 = p["layers"]
    r2 = lambda b: b.reshape(1, H)
    src = edge_index[0]
    dst = edge_index[1]

    wf1, bf1, wf2, bf2 = _fold_weights(
        p["emb_e_W"], p["emb_e_b"].reshape(1, H),
        l1["C_W"], r2(l1["C_b"]), l2["C_W"], r2(l2["C_b"]))

    hh = _embed(h, p["emb_h_W"], p["emb_h_b"].reshape(1, H))

    ah, dbt, eht = _node_matmuls(
        hh, l1["A_W"], r2(l1["A_b"]), l1["B_W"], r2(l1["B_b"]),
        l1["D_W"], r2(l1["D_b"]), l1["E_W"], r2(l1["E_b"]))
    ce1 = _edge_linear1(e, wf1, bf1)
    acc1, enew1 = _sc_pass1(
        ce1.reshape(2 * E, HH), dbt.reshape(2 * N, H), eht, src, dst)
    raw1, ps1, ps21 = _node_update_a(ah, acc1.reshape(2, N, H))
    hh1, _ = _node_update_b(raw1, ps1, ps21, r2(l1["bn_h_g"]),
                            r2(l1["bn_h_b"]), hh)

    ah2, dbt2, eht2 = _node_matmuls(
        hh1, l2["A_W"], r2(l2["A_b"]), l2["B_W"], r2(l2["B_b"]),
        l2["D_W"], r2(l2["D_b"]), l2["E_W"], r2(l2["E_b"]))
    eps1, eps2 = _edge_stats(enew1.reshape(2, E, HH))
    ce2 = _edge_linear2(e, enew1.reshape(2, E, HH), eps1, eps2, wf2, bf2,
                        l2["C_W"], r2(l1["bn_e_g"]), r2(l1["bn_e_b"]))
    (acc2,) = _sc_pass2(
        ce2.reshape(2 * E, HH), dbt2.reshape(2 * N, H), eht2, src, dst)
    raw2, ps2, ps22 = _node_update_a(ah2, acc2.reshape(2, N, H))
    hh2, psh = _node_update_b(raw2, ps2, ps22, r2(l2["bn_h_g"]),
                              r2(l2["bn_h_b"]), hh1)

    return _head(psh, state, p["l1_W"], p["l1_b"].reshape(1, 256),
                 p["l2_W"], p["l2_b"].reshape(1, 256),
                 p["l3_W"], p["l3_b"].reshape(1, 2))


# speedup vs baseline: 2.8011x; 1.0022x over previous
import functools

import jax
import jax.numpy as jnp
from jax import lax
from jax.experimental import pallas as pl
from jax.experimental.pallas import tpu as pltpu
from jax.experimental.pallas import tpu_sc as plsc

N = 10000
E = 320000
H = 128
HH = 64
NBLK = 1000
EBLK = 2000
B = 64
NUM_TILES = 16
DRAIN_ROWS = 624
ZROWS = 16
NUM_EBLK = E // B
F32 = jnp.float32


def _f32(x):
    return jax.ShapeDtypeStruct(x, F32)


def _fold_body(embW, embb, c1W, c1b, c2W, c2b, wf1, bf1, wf2, bf2):
    wf1[...] = jnp.dot(embW[...], c1W[...], preferred_element_type=F32)
    bf1[...] = jnp.dot(embb[...], c1W[...], preferred_element_type=F32) + c1b[...]
    wf2[...] = jnp.dot(embW[...], c2W[...], preferred_element_type=F32)
    bf2[...] = jnp.dot(embb[...], c2W[...], preferred_element_type=F32) + c2b[...]


def _fold_weights(embW, embb, c1W, c1b, c2W, c2b):
    return pl.pallas_call(
        _fold_body,
        out_shape=[_f32((16, H)), _f32((1, H)), _f32((16, H)), _f32((1, H))],
    )(embW, embb, c1W, c1b, c2W, c2b)


def _embed_body(h, w, b, out):
    out[...] = jnp.dot(h[...], w[...], preferred_element_type=F32) + b[...]


def _embed(h, w, b):
    return pl.pallas_call(
        _embed_body,
        grid=(N // NBLK,),
        in_specs=[
            pl.BlockSpec((NBLK, H), lambda i: (i, 0)),
            pl.BlockSpec((H, H), lambda i: (0, 0)),
            pl.BlockSpec((1, H), lambda i: (0, 0)),
        ],
        out_specs=pl.BlockSpec((NBLK, H), lambda i: (i, 0)),
        out_shape=_f32((N, H)),
    )(h, w, b)


def _mm_tables(x, aW, ab, bW, bb, dW, db_, eW, eb, ah, dbt, eht):
    ah[...] = jnp.dot(x, aW[...], preferred_element_type=F32) + ab[...]
    Bh = jnp.dot(x, bW[...], preferred_element_type=F32) + bb[...]
    Dh = jnp.dot(x, dW[...], preferred_element_type=F32) + db_[...]
    Eh = jnp.dot(x, eW[...], preferred_element_type=F32) + eb[...]
    dbt[0] = jnp.concatenate([Dh[:, :HH], Bh[:, :HH]], axis=1)
    dbt[1] = jnp.concatenate([Dh[:, HH:], Bh[:, HH:]], axis=1)
    eht[...] = Eh


def _node_first_body(h, embW, embb, aW, ab, bW, bb, dW, db_, eW, eb,
                     hh, ah, dbt, eht):
    x = jnp.dot(h[...], embW[...], preferred_element_type=F32) + embb[...]
    hh[...] = x
    _mm_tables(x, aW, ab, bW, bb, dW, db_, eW, eb, ah, dbt, eht)


def _node_first(h, embW, embb, aW, ab, bW, bb, dW, db_, eW, eb):
    return pl.pallas_call(
        _node_first_body,
        grid=(N // NBLK,),
        in_specs=[pl.BlockSpec((NBLK, H), lambda i: (i, 0))]
        + [pl.BlockSpec((H, H), lambda i: (0, 0)),
           pl.BlockSpec((1, H), lambda i: (0, 0))] * 5,
        out_specs=[
            pl.BlockSpec((NBLK, H), lambda i: (i, 0)),
            pl.BlockSpec((NBLK, H), lambda i: (i, 0)),
            pl.BlockSpec((2, NBLK, H), lambda i: (0, i, 0)),
            pl.BlockSpec((NBLK, H), lambda i: (i, 0)),
        ],
        out_shape=[_f32((N, H)), _f32((N, H)), _f32((2, N, H)), _f32((N, H))],
    )(h, embW, embb, aW, ab, bW, bb, dW, db_, eW, eb)


def _node_mid_body(raw, ps, ps2, g, b, hin, aW, ab, bW, bb, dW, db_, eW, eb,
                   hh, ah, dbt, eht):
    inv_n = 1.0 / float(N)
    m = jnp.sum(ps[...], axis=0) * inv_n
    v = jnp.sum(ps2[...], axis=0) * inv_n - m * m
    hn = (raw[...] - m) * lax.rsqrt(v + 1e-5) * g[...] + b[...]
    x = hin[...] + jnp.maximum(hn, 0.0)
    hh[...] = x
    _mm_tables(x, aW, ab, bW, bb, dW, db_, eW, eb, ah, dbt, eht)


def _node_mid(raw, ps, ps2, g, b, hin, aW, ab, bW, bb, dW, db_, eW, eb):
    return pl.pallas_call(
        _node_mid_body,
        grid=(N // NBLK,),
        in_specs=[
            pl.BlockSpec((NBLK, H), lambda i: (i, 0)),
            pl.BlockSpec((N // NBLK, 1, H), lambda i: (0, 0, 0)),
            pl.BlockSpec((N // NBLK, 1, H), lambda i: (0, 0, 0)),
            pl.BlockSpec((1, H), lambda i: (0, 0)),
            pl.BlockSpec((1, H), lambda i: (0, 0)),
            pl.BlockSpec((NBLK, H), lambda i: (i, 0)),
        ]
        + [pl.BlockSpec((H, H), lambda i: (0, 0)),
           pl.BlockSpec((1, H), lambda i: (0, 0))] * 4,
        out_specs=[
            pl.BlockSpec((NBLK, H), lambda i: (i, 0)),
            pl.BlockSpec((NBLK, H), lambda i: (i, 0)),
            pl.BlockSpec((2, NBLK, H), lambda i: (0, i, 0)),
            pl.BlockSpec((NBLK, H), lambda i: (i, 0)),
        ],
        out_shape=[_f32((N, H)), _f32((N, H)), _f32((2, N, H)), _f32((N, H))],
    )(raw, ps, ps2, g, b, hin, aW, ab, bW, bb, dW, db_, eW, eb)


def _edge_lin1_body(e, wf, bf, out):
    ce = jnp.dot(1.0 / e[...], wf[...], preferred_element_type=F32) + bf[...]
    out[0] = ce[:, :HH]
    out[1] = ce[:, HH:]


def _edge_linear1(e, wf, bf):
    return pl.pallas_call(
        _edge_lin1_body,
        grid=(E // EBLK,),
        in_specs=[
            pl.BlockSpec((EBLK, 16), lambda i: (i, 0)),
            pl.BlockSpec((16, H), lambda i: (0, 0)),
            pl.BlockSpec((1, H), lambda i: (0, 0)),
        ],
        out_specs=pl.BlockSpec((2, EBLK, HH), lambda i: (0, i, 0)),
        out_shape=_f32((2, E, HH)),
    )(e, wf, bf)


def _edge_stats_body(enew, ps, ps2):
    x0 = enew[0]
    x1 = enew[1]
    s1 = jnp.concatenate([jnp.sum(x0, axis=0, keepdims=True),
                          jnp.sum(x1, axis=0, keepdims=True)], axis=1)
    s2 = jnp.concatenate([jnp.sum(x0 * x0, axis=0, keepdims=True),
                          jnp.sum(x1 * x1, axis=0, keepdims=True)], axis=1)
    ps[...] = s1.reshape(1, 1, H)
    ps2[...] = s2.reshape(1, 1, H)


def _edge_stats(enew):
    return pl.pallas_call(
        _edge_stats_body,
        grid=(E // EBLK,),
        in_specs=[pl.BlockSpec((2, EBLK, HH), lambda i: (0, i, 0))],
        out_specs=[
            pl.BlockSpec((1, 1, H), lambda i: (i, 0, 0)),
            pl.BlockSpec((1, 1, H), lambda i: (i, 0, 0)),
        ],
        out_shape=[_f32((E // EBLK, 1, H)), _f32((E // EBLK, 1, H))],
    )(enew)


def _edge_lin2_body(e, enew, ps, ps2, wf, bf, cW, g, b, out):
    inv_e = 1.0 / float(E)
    s1 = jnp.sum(ps[...], axis=0)
    s2 = jnp.sum(ps2[...], axis=0)
    m0, m1 = s1[:, :HH] * inv_e, s1[:, HH:] * inv_e
    s2h0, s2h1 = s2[:, :HH], s2[:, HH:]
    v0 = s2h0 * inv_e - m0 * m0
    v1 = s2h1 * inv_e - m1 * m1
    r0 = (enew[0] - m0) * lax.rsqrt(v0 + 1e-5) * g[:, :HH] + b[:, :HH]
    r1 = (enew[1] - m1) * lax.rsqrt(v1 + 1e-5) * g[:, HH:] + b[:, HH:]
    r0 = jnp.maximum(r0, 0.0).astype(jnp.bfloat16)
    r1 = jnp.maximum(r1, 0.0).astype(jnp.bfloat16)
    cw16 = cW[...].astype(jnp.bfloat16)
    ce = (jnp.dot(1.0 / e[...], wf[...], preferred_element_type=F32) + bf[...]
          + jnp.dot(r0, cw16[:HH, :], preferred_element_type=F32)
          + jnp.dot(r1, cw16[HH:, :], preferred_element_type=F32))
    out[0] = ce[:, :HH]
    out[1] = ce[:, HH:]


def _edge_linear2(e, enew, ps, ps2, wf, bf, cW, g, b):
    return pl.pallas_call(
        _edge_lin2_body,
        grid=(E // EBLK,),
        in_specs=[
            pl.BlockSpec((EBLK, 16), lambda i: (i, 0)),
            pl.BlockSpec((2, EBLK, HH), lambda i: (0, i, 0)),
            pl.BlockSpec((E // EBLK, 1, H), lambda i: (0, 0, 0)),
            pl.BlockSpec((E // EBLK, 1, H), lambda i: (0, 0, 0)),
            pl.BlockSpec((16, H), lambda i: (0, 0)),
            pl.BlockSpec((1, H), lambda i: (0, 0)),
            pl.BlockSpec((H, H), lambda i: (0, 0)),
            pl.BlockSpec((1, H), lambda i: (0, 0)),
            pl.BlockSpec((1, H), lambda i: (0, 0)),
        ],
        out_specs=pl.BlockSpec((2, EBLK, HH), lambda i: (0, i, 0)),
        out_shape=_f32((2, E, HH)),
    )(e, enew, ps, ps2, wf, bf, cW, g, b)


def _node_upd_a_body(ah, acc, raw, ps, ps2):
    num = jnp.concatenate([acc[0, :, :HH], acc[1, :, :HH]], axis=1)
    den = jnp.concatenate([acc[0, :, HH:], acc[1, :, HH:]], axis=1)
    r = ah[...] + num / (den + 1e-6)
    raw[...] = r
    ps[...] = jnp.sum(r, axis=0, keepdims=True).reshape(1, 1, H)
    ps2[...] = jnp.sum(r * r, axis=0, keepdims=True).reshape(1, 1, H)


def _node_update_a(ah, acc):
    return pl.pallas_call(
        _node_upd_a_body,
        grid=(N // NBLK,),
        in_specs=[
            pl.BlockSpec((NBLK, H), lambda i: (i, 0)),
            pl.BlockSpec((2, NBLK, H), lambda i: (0, i, 0)),
        ],
        out_specs=[
            pl.BlockSpec((NBLK, H), lambda i: (i, 0)),
            pl.BlockSpec((1, 1, H), lambda i: (i, 0, 0)),
            pl.BlockSpec((1, 1, H), lambda i: (i, 0, 0)),
        ],
        out_shape=[_f32((N, H)), _f32((N // NBLK, 1, H)), _f32((N // NBLK, 1, H))],
    )(ah, acc)


def _node_upd_b_body(raw, ps, ps2, g, b, hin, out, psh):
    inv_n = 1.0 / float(N)
    m = jnp.sum(ps[...], axis=0) * inv_n
    v = jnp.sum(ps2[...], axis=0) * inv_n - m * m
    hn = (raw[...] - m) * lax.rsqrt(v + 1e-5) * g[...] + b[...]
    r = hin[...] + jnp.maximum(hn, 0.0)
    out[...] = r
    psh[...] = jnp.sum(r, axis=0, keepdims=True).reshape(1, 1, H)


def _node_update_b(raw, ps, ps2, g, b, hin):
    return pl.pallas_call(
        _node_upd_b_body,
        grid=(N // NBLK,),
        in_specs=[
            pl.BlockSpec((NBLK, H), lambda i: (i, 0)),
            pl.BlockSpec((N // NBLK, 1, H), lambda i: (0, 0, 0)),
            pl.BlockSpec((N // NBLK, 1, H), lambda i: (0, 0, 0)),
            pl.BlockSpec((1, H), lambda i: (0, 0)),
            pl.BlockSpec((1, H), lambda i: (0, 0)),
            pl.BlockSpec((NBLK, H), lambda i: (i, 0)),
        ],
        out_specs=[
            pl.BlockSpec((NBLK, H), lambda i: (i, 0)),
            pl.BlockSpec((1, 1, H), lambda i: (i, 0, 0)),
        ],
        out_shape=[_f32((N, H)), _f32((N // NBLK, 1, H))],
    )(raw, ps, ps2, g, b, hin)


def _head_body(psh, st, w1, b1, w2, b2, w3, b3, out):
    hm = jnp.sum(psh[...], axis=0) * (1.0 / float(N))
    z = jnp.concatenate([hm, st[...]], axis=1)
    z = jnp.maximum(jnp.dot(z, w1[...], preferred_element_type=F32) + b1[...], 0.0)
    z = jnp.maximum(jnp.dot(z, w2[...], preferred_element_type=F32) + b2[...], 0.0)
    out[...] = jnp.tanh(jnp.dot(z, w3[...], preferred_element_type=F32) + b3[...])


def _head(psh, st, w1, b1, w2, b2, w3, b3):
    return pl.pallas_call(_head_body, out_shape=_f32((1, 2)))(
        psh, st, w1, b1, w2, b2, w3, b3)


def _sigmoid16(x):
    return 1.0 / (1.0 + jnp.exp(-x))


def _sc_body(do_enew, *refs):
    if do_enew:
        (ce, dbt, eht, src, dst, accout, enew,
         src_v0, src_v1, dst_v0, dst_v1, dsts_v0, dsts_v1,
         ce_v0, ce_v1, db_v0, db_v1, eh_v0, eh_v1,
         s_src0, s_src1, s_dst0, s_dst1, s_ce0, s_ce1,
         s_gdb0, s_gdb1, s_geh0, s_geh1, s_scat, s_en0, s_en1,
         acc_sh) = refs
        s_en = [s_en0, s_en1]
    else:
        (ce, dbt, eht, src, dst, accout,
         src_v0, src_v1, dst_v0, dst_v1, dsts_v0, dsts_v1,
         ce_v0, ce_v1, db_v0, db_v1, eh_v0, eh_v1,
         s_src0, s_src1, s_dst0, s_dst1, s_ce0, s_ce1,
         s_gdb0, s_gdb1, s_geh0, s_geh1, s_scat,
         acc_sh) = refs
    src_vs = [src_v0, src_v1]
    dst_vs = [dst_v0, dst_v1]
    dsts_vs = [dsts_v0, dsts_v1]
    ce_vs = [ce_v0, ce_v1]
    db_vs = [db_v0, db_v1]
    eh_vs = [eh_v0, eh_v1]
    s_src = [s_src0, s_src1]
    s_dst = [s_dst0, s_dst1]
    s_ce = [s_ce0, s_ce1]
    s_gdb = [s_gdb0, s_gdb1]
    s_geh = [s_geh0, s_geh1]

    cid = lax.axis_index("c")
    sid = lax.axis_index("s")

    def zfill(i, _):
        for c in range(H // 16):
            db_v0[i, pl.ds(c * 16, 16)] = jnp.zeros((16,), F32)
        return 0
    lax.fori_loop(0, ZROWS, zfill, 0)
    base = sid * DRAIN_ROWS
    zsrc = db_v0.at[pl.ds(0, ZROWS)]

    def zcopy(k, _):
        pltpu.sync_copy(zsrc, acc_sh.at[pl.ds(base + k * ZROWS, ZROWS)])
        return 0
    lax.fori_loop(0, DRAIN_ROWS // ZROWS, zcopy, 0)

    @pl.when(sid == NUM_TILES - 1)
    def _():
        pltpu.sync_copy(zsrc, acc_sh.at[pl.ds(N - ZROWS, ZROWS)])
    plsc.subcore_barrier()

    nblk = (NUM_EBLK - sid + NUM_TILES - 1) // NUM_TILES
    tbl_off = cid * N

    def _off(j):
        return (sid + j * NUM_TILES) * B

    def idx_issue(slot, j):
        o = _off(j)
        pltpu.async_copy(src.at[pl.ds(o, B)], src_vs[slot], s_src[slot])
        pltpu.async_copy(dst.at[pl.ds(o, B)], dst_vs[slot], s_dst[slot])

    def idx_wait(slot):
        pltpu.make_async_copy(src.at[pl.ds(0, B)], src_vs[slot], s_src[slot]).wait()
        pltpu.make_async_copy(dst.at[pl.ds(0, B)], dst_vs[slot], s_dst[slot]).wait()

    def ce_issue(slot, j):
        pltpu.async_copy(ce.at[pl.ds(cid * E + _off(j), B)], ce_vs[slot],
                         s_ce[slot])

    def ce_wait(slot):
        pltpu.make_async_copy(ce.at[pl.ds(0, B)], ce_vs[slot], s_ce[slot]).wait()

    def adjust(slot):
        for c in range(B // 16):
            cs = pl.ds(c * 16, 16)
            src_vs[slot][cs] = src_vs[slot][cs] + tbl_off
            dsts_vs[slot][cs] = dst_vs[slot][cs]

    def gather_issue(slot):
        pltpu.async_copy(dbt.at[src_vs[slot]], db_vs[slot], s_gdb[slot])
        pltpu.async_copy(eht.at[dsts_vs[slot]], eh_vs[slot], s_geh[slot])

    def gather_wait(slot):
        pltpu.make_async_copy(dbt.at[src_vs[slot]], db_vs[slot],
                              s_gdb[slot]).wait()
        pltpu.make_async_copy(eht.at[dsts_vs[slot]], eh_vs[slot],
                              s_geh[slot]).wait()

    def scat_wait():
        pltpu.make_async_copy(db_v0, acc_sh.at[dsts_v0], s_scat).wait()

    def enew_wait(slot):
        pltpu.make_async_copy(ce_vs[slot], enew.at[pl.ds(0, B)], s_en[slot]).wait()

    def block(j, slot, _carry):
        oth = 1 - slot

        @pl.when(j + 1 < nblk)
        def _():
            idx_wait(oth)

        @pl.when(j >= 1)
        def _():
            scat_wait()
        if do_enew:
            @pl.when(j >= 1)
            def _():
                enew_wait(oth)

        @pl.when(j + 1 < nblk)
        def _():
            ce_issue(oth, j + 1)
            adjust(oth)
            gather_issue(oth)

        gather_wait(slot)
        ce_wait(slot)

        @pl.when(j + 2 < nblk)
        def _():
            idx_issue(slot, j + 2)

        @plsc.parallel_loop(0, B, 1, unroll=4)
        def edge(e_i):
            for c in range(HH // 16):
                ds = pl.ds(c * 16, 16)
                dsb = pl.ds(HH + c * 16, 16)
                en = (ce_vs[slot][e_i, ds] + db_vs[slot][e_i, ds]
                      + eh_vs[slot][e_i, pl.ds(cid * HH + c * 16, 16)])
                if do_enew:
                    ce_vs[slot][e_i, ds] = en
                sg = _sigmoid16(en)
                num = sg * db_vs[slot][e_i, dsb]
                db_vs[slot][e_i, ds] = num
                db_vs[slot][e_i, dsb] = sg

        if do_enew:
            pltpu.async_copy(ce_vs[slot], enew.at[pl.ds(cid * E + _off(j), B)],
                             s_en[slot])
        pltpu.async_copy(db_vs[slot], acc_sh.at[dsts_vs[slot]], s_scat, add=True)
        return 0

    idx_issue(0, 0)
    idx_wait(0)
    ce_issue(0, 0)
    adjust(0)
    gather_issue(0)
    idx_issue(1, 1)

    npair = nblk // 2

    def pair(g, carry):
        block(2 * g, 0, carry)
        return block(2 * g + 1, 1, carry)

    lax.fori_loop(0, npair, pair, 0)
    lax.fori_loop(2 * npair, nblk, lambda j, c_: block(j, 0, c_), 0)

    scat_wait()
    if do_enew:
        @pl.when(nblk % 2 == 1)
        def _():
            enew_wait(0)

        @pl.when(nblk % 2 == 0)
        def _():
            enew_wait(1)

    plsc.subcore_barrier()
    pltpu.sync_copy(
        acc_sh.at[pl.ds(base, DRAIN_ROWS)],
        accout.at[pl.ds(cid * N + base, DRAIN_ROWS)])

    @pl.when(sid == NUM_TILES - 1)
    def _():
        pltpu.sync_copy(
            acc_sh.at[pl.ds(N - ZROWS, ZROWS)],
            accout.at[pl.ds(cid * N + N - ZROWS, ZROWS)])


def _make_sc_edge_pass(do_enew):
    out_type = [_f32((2 * N, H))]
    if do_enew:
        out_type += [_f32((2 * E, HH))]
    scratch = [
        pltpu.VMEM((B,), jnp.int32),
        pltpu.VMEM((B,), jnp.int32),
        pltpu.VMEM((B,), jnp.int32),
        pltpu.VMEM((B,), jnp.int32),
        pltpu.VMEM((B,), jnp.int32),
        pltpu.VMEM((B,), jnp.int32),
        pltpu.VMEM((B, HH), F32),
        pltpu.VMEM((B, HH), F32),
        pltpu.VMEM((B, H), F32),
        pltpu.VMEM((B, H), F32),
        pltpu.VMEM((B, H), F32),
        pltpu.VMEM((B, H), F32),
        pltpu.SemaphoreType.DMA,
        pltpu.SemaphoreType.DMA,
        pltpu.SemaphoreType.DMA,
        pltpu.SemaphoreType.DMA,
        pltpu.SemaphoreType.DMA,
        pltpu.SemaphoreType.DMA,
        pltpu.SemaphoreType.DMA,
        pltpu.SemaphoreType.DMA,
        pltpu.SemaphoreType.DMA,
        pltpu.SemaphoreType.DMA,
        pltpu.SemaphoreType.DMA,
    ]
    if do_enew:
        scratch += [pltpu.SemaphoreType.DMA, pltpu.SemaphoreType.DMA]
    scratch += [pltpu.VMEM_SHARED((N, H), F32)]
    return pl.kernel(
        functools.partial(_sc_body, do_enew),
        out_type=out_type,
        mesh=plsc.VectorSubcoreMesh(core_axis_name="c", subcore_axis_name="s"),
        scratch_types=scratch,
    )


_sc_pass1 = _make_sc_edge_pass(True)
_sc_pass2 = _make_sc_edge_pass(False)


def kernel(h, e, state, params, edge_index):
    p = params
    l1, l2 = p["layers"]
    r2 = lambda b: b.reshape(1, H)
    src = edge_index[0]
    dst = edge_index[1]

    wf1, bf1, wf2, bf2 = _fold_weights(
        p["emb_e_W"], p["emb_e_b"].reshape(1, H),
        l1["C_W"], r2(l1["C_b"]), l2["C_W"], r2(l2["C_b"]))

    hh, ah, dbt, eht = _node_first(
        h, p["emb_h_W"], p["emb_h_b"].reshape(1, H),
        l1["A_W"], r2(l1["A_b"]), l1["B_W"], r2(l1["B_b"]),
        l1["D_W"], r2(l1["D_b"]), l1["E_W"], r2(l1["E_b"]))
    ce1 = _edge_linear1(e, wf1, bf1)
    acc1, enew1 = _sc_pass1(
        ce1.reshape(2 * E, HH), dbt.reshape(2 * N, H), eht, src, dst)
    raw1, ps1, ps21 = _node_update_a(ah, acc1.reshape(2, N, H))

    hh1, ah2, dbt2, eht2 = _node_mid(
        raw1, ps1, ps21, r2(l1["bn_h_g"]), r2(l1["bn_h_b"]), hh,
        l2["A_W"], r2(l2["A_b"]), l2["B_W"], r2(l2["B_b"]),
        l2["D_W"], r2(l2["D_b"]), l2["E_W"], r2(l2["E_b"]))
    eps1, eps2 = _edge_stats(enew1.reshape(2, E, HH))
    ce2 = _edge_linear2(e, enew1.reshape(2, E, HH), eps1, eps2, wf2, bf2,
                        l2["C_W"], r2(l1["bn_e_g"]), r2(l1["bn_e_b"]))
    (acc2,) = _sc_pass2(
        ce2.reshape(2 * E, HH), dbt2.reshape(2 * N, H), eht2, src, dst)
    raw2, ps2, ps22 = _node_update_a(ah2, acc2.reshape(2, N, H))
    hh2, psh = _node_update_b(raw2, ps2, ps22, r2(l2["bn_h_g"]),
                              r2(l2["bn_h_b"]), hh1)

    return _head(psh, state, p["l1_W"], p["l1_b"].reshape(1, 256),
                 p["l2_W"], p["l2_b"].reshape(1, 256),
                 p["l3_W"], p["l3_b"].reshape(1, 2))


# speedup vs baseline: 3.0457x; 1.0873x over previous
import functools

import jax
import jax.numpy as jnp
from jax import lax
from jax.experimental import pallas as pl
from jax.experimental.pallas import tpu as pltpu
from jax.experimental.pallas import tpu_sc as plsc

N = 10000
E = 320000
H = 128
HH = 64
NBLK = 2000
EBLK = 4000
B = 64
NUM_TILES = 16
DRAIN_ROWS = 624
ZROWS = 16
NUM_EBLK = E // B
F32 = jnp.float32


def _f32(x):
    return jax.ShapeDtypeStruct(x, F32)


def _fold_body(embW, embb, c1W, c1b, c2W, c2b, wf1, bf1, wf2, bf2):
    wf1[...] = jnp.dot(embW[...], c1W[...], preferred_element_type=F32)
    bf1[...] = jnp.dot(embb[...], c1W[...], preferred_element_type=F32) + c1b[...]
    wf2[...] = jnp.dot(embW[...], c2W[...], preferred_element_type=F32)
    bf2[...] = jnp.dot(embb[...], c2W[...], preferred_element_type=F32) + c2b[...]


def _fold_weights(embW, embb, c1W, c1b, c2W, c2b):
    return pl.pallas_call(
        _fold_body,
        out_shape=[_f32((16, H)), _f32((1, H)), _f32((16, H)), _f32((1, H))],
    )(embW, embb, c1W, c1b, c2W, c2b)


def _embed_body(h, w, b, out):
    out[...] = jnp.dot(h[...], w[...], preferred_element_type=F32) + b[...]


def _embed(h, w, b):
    return pl.pallas_call(
        _embed_body,
        grid=(N // NBLK,),
        in_specs=[
            pl.BlockSpec((NBLK, H), lambda i: (i, 0)),
            pl.BlockSpec((H, H), lambda i: (0, 0)),
            pl.BlockSpec((1, H), lambda i: (0, 0)),
        ],
        out_specs=pl.BlockSpec((NBLK, H), lambda i: (i, 0)),
        out_shape=_f32((N, H)),
    )(h, w, b)


def _mm_tables(x, aW, ab, bW, bb, dW, db_, eW, eb, ah, dbt, eht):
    ah[...] = jnp.dot(x, aW[...], preferred_element_type=F32) + ab[...]
    Bh = jnp.dot(x, bW[...], preferred_element_type=F32) + bb[...]
    Dh = jnp.dot(x, dW[...], preferred_element_type=F32) + db_[...]
    Eh = jnp.dot(x, eW[...], preferred_element_type=F32) + eb[...]
    dbt[0] = jnp.concatenate([Dh[:, :HH], Bh[:, :HH]], axis=1)
    dbt[1] = jnp.concatenate([Dh[:, HH:], Bh[:, HH:]], axis=1)
    eht[...] = Eh


def _node_first_body(h, embW, embb, aW, ab, bW, bb, dW, db_, eW, eb,
                     hh, ah, dbt, eht):
    x = jnp.dot(h[...], embW[...], preferred_element_type=F32) + embb[...]
    hh[...] = x
    _mm_tables(x, aW, ab, bW, bb, dW, db_, eW, eb, ah, dbt, eht)


def _node_first(h, embW, embb, aW, ab, bW, bb, dW, db_, eW, eb):
    return pl.pallas_call(
        _node_first_body,
        grid=(N // NBLK,),
        in_specs=[pl.BlockSpec((NBLK, H), lambda i: (i, 0))]
        + [pl.BlockSpec((H, H), lambda i: (0, 0)),
           pl.BlockSpec((1, H), lambda i: (0, 0))] * 5,
        out_specs=[
            pl.BlockSpec((NBLK, H), lambda i: (i, 0)),
            pl.BlockSpec((NBLK, H), lambda i: (i, 0)),
            pl.BlockSpec((2, NBLK, H), lambda i: (0, i, 0)),
            pl.BlockSpec((NBLK, H), lambda i: (i, 0)),
        ],
        out_shape=[_f32((N, H)), _f32((N, H)), _f32((2, N, H)), _f32((N, H))],
    )(h, embW, embb, aW, ab, bW, bb, dW, db_, eW, eb)


def _node_mid_body(raw, ps, ps2, g, b, hin, aW, ab, bW, bb, dW, db_, eW, eb,
                   hh, ah, dbt, eht):
    inv_n = 1.0 / float(N)
    m = jnp.sum(ps[...], axis=0) * inv_n
    v = jnp.sum(ps2[...], axis=0) * inv_n - m * m
    hn = (raw[...] - m) * lax.rsqrt(v + 1e-5) * g[...] + b[...]
    x = hin[...] + jnp.maximum(hn, 0.0)
    hh[...] = x
    _mm_tables(x, aW, ab, bW, bb, dW, db_, eW, eb, ah, dbt, eht)


def _node_mid(raw, ps, ps2, g, b, hin, aW, ab, bW, bb, dW, db_, eW, eb):
    return pl.pallas_call(
        _node_mid_body,
        grid=(N // NBLK,),
        in_specs=[
            pl.BlockSpec((NBLK, H), lambda i: (i, 0)),
            pl.BlockSpec((N // NBLK, 1, H), lambda i: (0, 0, 0)),
            pl.BlockSpec((N // NBLK, 1, H), lambda i: (0, 0, 0)),
            pl.BlockSpec((1, H), lambda i: (0, 0)),
            pl.BlockSpec((1, H), lambda i: (0, 0)),
            pl.BlockSpec((NBLK, H), lambda i: (i, 0)),
        ]
        + [pl.BlockSpec((H, H), lambda i: (0, 0)),
           pl.BlockSpec((1, H), lambda i: (0, 0))] * 4,
        out_specs=[
            pl.BlockSpec((NBLK, H), lambda i: (i, 0)),
            pl.BlockSpec((NBLK, H), lambda i: (i, 0)),
            pl.BlockSpec((2, NBLK, H), lambda i: (0, i, 0)),
            pl.BlockSpec((NBLK, H), lambda i: (i, 0)),
        ],
        out_shape=[_f32((N, H)), _f32((N, H)), _f32((2, N, H)), _f32((N, H))],
    )(raw, ps, ps2, g, b, hin, aW, ab, bW, bb, dW, db_, eW, eb)


def _edge_lin1_body(e, wf, bf, out):
    ce = jnp.dot(1.0 / e[...], wf[...], preferred_element_type=F32) + bf[...]
    out[0] = ce[:, :HH]
    out[1] = ce[:, HH:]


def _edge_linear1(e, wf, bf):
    return pl.pallas_call(
        _edge_lin1_body,
        grid=(E // EBLK,),
        in_specs=[
            pl.BlockSpec((EBLK, 16), lambda i: (i, 0)),
            pl.BlockSpec((16, H), lambda i: (0, 0)),
            pl.BlockSpec((1, H), lambda i: (0, 0)),
        ],
        out_specs=pl.BlockSpec((2, EBLK, HH), lambda i: (0, i, 0)),
        out_shape=_f32((2, E, HH)),
    )(e, wf, bf)


def _edge_stats_body(enew, ps, ps2):
    x0 = enew[0]
    x1 = enew[1]
    s1 = jnp.concatenate([jnp.sum(x0, axis=0, keepdims=True),
                          jnp.sum(x1, axis=0, keepdims=True)], axis=1)
    s2 = jnp.concatenate([jnp.sum(x0 * x0, axis=0, keepdims=True),
                          jnp.sum(x1 * x1, axis=0, keepdims=True)], axis=1)
    ps[...] = s1.reshape(1, 1, H)
    ps2[...] = s2.reshape(1, 1, H)


def _edge_stats(enew):
    return pl.pallas_call(
        _edge_stats_body,
        grid=(E // EBLK,),
        in_specs=[pl.BlockSpec((2, EBLK, HH), lambda i: (0, i, 0))],
        out_specs=[
            pl.BlockSpec((1, 1, H), lambda i: (i, 0, 0)),
            pl.BlockSpec((1, 1, H), lambda i: (i, 0, 0)),
        ],
        out_shape=[_f32((E // EBLK, 1, H)), _f32((E // EBLK, 1, H))],
    )(enew)


def _edge_lin2_body(e, enew, ps, ps2, wf, bf, cW, g, b, out):
    inv_e = 1.0 / float(E)
    s1 = jnp.sum(ps[...], axis=0)
    s2 = jnp.sum(ps2[...], axis=0)
    m0, m1 = s1[:, :HH] * inv_e, s1[:, HH:] * inv_e
    s2h0, s2h1 = s2[:, :HH], s2[:, HH:]
    v0 = s2h0 * inv_e - m0 * m0
    v1 = s2h1 * inv_e - m1 * m1
    r0 = (enew[0] - m0) * lax.rsqrt(v0 + 1e-5) * g[:, :HH] + b[:, :HH]
    r1 = (enew[1] - m1) * lax.rsqrt(v1 + 1e-5) * g[:, HH:] + b[:, HH:]
    r0 = jnp.maximum(r0, 0.0).astype(jnp.bfloat16)
    r1 = jnp.maximum(r1, 0.0).astype(jnp.bfloat16)
    cw16 = cW[...].astype(jnp.bfloat16)
    ce = (jnp.dot(1.0 / e[...], wf[...], preferred_element_type=F32) + bf[...]
          + jnp.dot(r0, cw16[:HH, :], preferred_element_type=F32)
          + jnp.dot(r1, cw16[HH:, :], preferred_element_type=F32))
    out[0] = ce[:, :HH]
    out[1] = ce[:, HH:]


def _edge_linear2(e, enew, ps, ps2, wf, bf, cW, g, b):
    return pl.pallas_call(
        _edge_lin2_body,
        grid=(E // EBLK,),
        in_specs=[
            pl.BlockSpec((EBLK, 16), lambda i: (i, 0)),
            pl.BlockSpec((2, EBLK, HH), lambda i: (0, i, 0)),
            pl.BlockSpec((E // EBLK, 1, H), lambda i: (0, 0, 0)),
            pl.BlockSpec((E // EBLK, 1, H), lambda i: (0, 0, 0)),
            pl.BlockSpec((16, H), lambda i: (0, 0)),
            pl.BlockSpec((1, H), lambda i: (0, 0)),
            pl.BlockSpec((H, H), lambda i: (0, 0)),
            pl.BlockSpec((1, H), lambda i: (0, 0)),
            pl.BlockSpec((1, H), lambda i: (0, 0)),
        ],
        out_specs=pl.BlockSpec((2, EBLK, HH), lambda i: (0, i, 0)),
        out_shape=_f32((2, E, HH)),
    )(e, enew, ps, ps2, wf, bf, cW, g, b)


def _node_upd_a_body(ah, acc, raw, ps, ps2):
    num = jnp.concatenate([acc[0, :, :HH], acc[1, :, :HH]], axis=1)
    den = jnp.concatenate([acc[0, :, HH:], acc[1, :, HH:]], axis=1)
    r = ah[...] + num / (den + 1e-6)
    raw[...] = r
    ps[...] = jnp.sum(r, axis=0, keepdims=True).reshape(1, 1, H)
    ps2[...] = jnp.sum(r * r, axis=0, keepdims=True).reshape(1, 1, H)


def _node_update_a(ah, acc):
    return pl.pallas_call(
        _node_upd_a_body,
        grid=(N // NBLK,),
        in_specs=[
            pl.BlockSpec((NBLK, H), lambda i: (i, 0)),
            pl.BlockSpec((2, NBLK, H), lambda i: (0, i, 0)),
        ],
        out_specs=[
            pl.BlockSpec((NBLK, H), lambda i: (i, 0)),
            pl.BlockSpec((1, 1, H), lambda i: (i, 0, 0)),
            pl.BlockSpec((1, 1, H), lambda i: (i, 0, 0)),
        ],
        out_shape=[_f32((N, H)), _f32((N // NBLK, 1, H)), _f32((N // NBLK, 1, H))],
    )(ah, acc)


def _node_upd_b_body(raw, ps, ps2, g, b, hin, out, psh):
    inv_n = 1.0 / float(N)
    m = jnp.sum(ps[...], axis=0) * inv_n
    v = jnp.sum(ps2[...], axis=0) * inv_n - m * m
    hn = (raw[...] - m) * lax.rsqrt(v + 1e-5) * g[...] + b[...]
    r = hin[...] + jnp.maximum(hn, 0.0)
    out[...] = r
    psh[...] = jnp.sum(r, axis=0, keepdims=True).reshape(1, 1, H)


def _node_update_b(raw, ps, ps2, g, b, hin):
    return pl.pallas_call(
        _node_upd_b_body,
        grid=(N // NBLK,),
        in_specs=[
            pl.BlockSpec((NBLK, H), lambda i: (i, 0)),
            pl.BlockSpec((N // NBLK, 1, H), lambda i: (0, 0, 0)),
            pl.BlockSpec((N // NBLK, 1, H), lambda i: (0, 0, 0)),
            pl.BlockSpec((1, H), lambda i: (0, 0)),
            pl.BlockSpec((1, H), lambda i: (0, 0)),
            pl.BlockSpec((NBLK, H), lambda i: (i, 0)),
        ],
        out_specs=[
            pl.BlockSpec((NBLK, H), lambda i: (i, 0)),
            pl.BlockSpec((1, 1, H), lambda i: (i, 0, 0)),
        ],
        out_shape=[_f32((N, H)), _f32((N // NBLK, 1, H))],
    )(raw, ps, ps2, g, b, hin)


def _head_body(psh, st, w1, b1, w2, b2, w3, b3, out):
    hm = jnp.sum(psh[...], axis=0) * (1.0 / float(N))
    z = jnp.concatenate([hm, st[...]], axis=1)
    z = jnp.maximum(jnp.dot(z, w1[...], preferred_element_type=F32) + b1[...], 0.0)
    z = jnp.maximum(jnp.dot(z, w2[...], preferred_element_type=F32) + b2[...], 0.0)
    out[...] = jnp.tanh(jnp.dot(z, w3[...], preferred_element_type=F32) + b3[...])


def _head(psh, st, w1, b1, w2, b2, w3, b3):
    return pl.pallas_call(_head_body, out_shape=_f32((1, 2)))(
        psh, st, w1, b1, w2, b2, w3, b3)


def _sigmoid16(x):
    return 1.0 / (1.0 + jnp.exp(-x))


def _sc_body(do_enew, *refs):
    if do_enew:
        (ce, dbt, eht, src, dst, accout, enew,
         src_v0, src_v1, dst_v0, dst_v1, dsts_v0, dsts_v1,
         ce_v0, ce_v1, db_v0, db_v1, eh_v0, eh_v1,
         s_src0, s_src1, s_dst0, s_dst1, s_ce0, s_ce1,
         s_gdb0, s_gdb1, s_geh0, s_geh1, s_scat, s_en0, s_en1,
         acc_sh) = refs
        s_en = [s_en0, s_en1]
    else:
        (ce, dbt, eht, src, dst, accout,
         src_v0, src_v1, dst_v0, dst_v1, dsts_v0, dsts_v1,
         ce_v0, ce_v1, db_v0, db_v1, eh_v0, eh_v1,
         s_src0, s_src1, s_dst0, s_dst1, s_ce0, s_ce1,
         s_gdb0, s_gdb1, s_geh0, s_geh1, s_scat,
         acc_sh) = refs
    src_vs = [src_v0, src_v1]
    dst_vs = [dst_v0, dst_v1]
    dsts_vs = [dsts_v0, dsts_v1]
    ce_vs = [ce_v0, ce_v1]
    db_vs = [db_v0, db_v1]
    eh_vs = [eh_v0, eh_v1]
    s_src = [s_src0, s_src1]
    s_dst = [s_dst0, s_dst1]
    s_ce = [s_ce0, s_ce1]
    s_gdb = [s_gdb0, s_gdb1]
    s_geh = [s_geh0, s_geh1]

    cid = lax.axis_index("c")
    sid = lax.axis_index("s")

    def zfill(i, _):
        for c in range(H // 16):
            db_v0[i, pl.ds(c * 16, 16)] = jnp.zeros((16,), F32)
        return 0
    lax.fori_loop(0, ZROWS, zfill, 0)
    base = sid * DRAIN_ROWS
    zsrc = db_v0.at[pl.ds(0, ZROWS)]

    def zcopy(k, _):
        pltpu.sync_copy(zsrc, acc_sh.at[pl.ds(base + k * ZROWS, ZROWS)])
        return 0
    lax.fori_loop(0, DRAIN_ROWS // ZROWS, zcopy, 0)

    @pl.when(sid == NUM_TILES - 1)
    def _():
        pltpu.sync_copy(zsrc, acc_sh.at[pl.ds(N - ZROWS, ZROWS)])
    plsc.subcore_barrier()

    nblk = (NUM_EBLK - sid + NUM_TILES - 1) // NUM_TILES
    tbl_off = cid * N

    def _off(j):
        return (sid + j * NUM_TILES) * B

    def idx_issue(slot, j):
        o = _off(j)
        pltpu.async_copy(src.at[pl.ds(o, B)], src_vs[slot], s_src[slot])
        pltpu.async_copy(dst.at[pl.ds(o, B)], dst_vs[slot], s_dst[slot])

    def idx_wait(slot):
        pltpu.make_async_copy(src.at[pl.ds(0, B)], src_vs[slot], s_src[slot]).wait()
        pltpu.make_async_copy(dst.at[pl.ds(0, B)], dst_vs[slot], s_dst[slot]).wait()

    def ce_issue(slot, j):
        pltpu.async_copy(ce.at[pl.ds(cid * E + _off(j), B)], ce_vs[slot],
                         s_ce[slot])

    def ce_wait(slot):
        pltpu.make_async_copy(ce.at[pl.ds(0, B)], ce_vs[slot], s_ce[slot]).wait()

    def adjust(slot):
        for c in range(B // 16):
            cs = pl.ds(c * 16, 16)
            src_vs[slot][cs] = src_vs[slot][cs] + tbl_off
            dsts_vs[slot][cs] = dst_vs[slot][cs]

    def gather_issue(slot):
        pltpu.async_copy(dbt.at[src_vs[slot]], db_vs[slot], s_gdb[slot])
        pltpu.async_copy(eht.at[dsts_vs[slot]], eh_vs[slot], s_geh[slot])

    def gather_wait(slot):
        pltpu.make_async_copy(dbt.at[src_vs[slot]], db_vs[slot],
                              s_gdb[slot]).wait()
        pltpu.make_async_copy(eht.at[dsts_vs[slot]], eh_vs[slot],
                              s_geh[slot]).wait()

    def scat_wait():
        pltpu.make_async_copy(db_v0, acc_sh.at[dsts_v0], s_scat).wait()

    def enew_wait(slot):
        pltpu.make_async_copy(ce_vs[slot], enew.at[pl.ds(0, B)], s_en[slot]).wait()

    def block(j, slot, _carry):
        oth = 1 - slot

        @pl.when(j + 1 < nblk)
        def _():
            idx_wait(oth)

        @pl.when(j >= 1)
        def _():
            scat_wait()
        if do_enew:
            @pl.when(j >= 1)
            def _():
                enew_wait(oth)

        @pl.when(j + 1 < nblk)
        def _():
            ce_issue(oth, j + 1)
            adjust(oth)
            gather_issue(oth)

        gather_wait(slot)
        ce_wait(slot)

        @pl.when(j + 2 < nblk)
        def _():
            idx_issue(slot, j + 2)

        @plsc.parallel_loop(0, B, 1, unroll=4)
        def edge(e_i):
            for c in range(HH // 16):
                ds = pl.ds(c * 16, 16)
                dsb = pl.ds(HH + c * 16, 16)
                en = (ce_vs[slot][e_i, ds] + db_vs[slot][e_i, ds]
                      + eh_vs[slot][e_i, pl.ds(cid * HH + c * 16, 16)])
                if do_enew:
                    ce_vs[slot][e_i, ds] = en
                sg = _sigmoid16(en)
                num = sg * db_vs[slot][e_i, dsb]
                db_vs[slot][e_i, ds] = num
                db_vs[slot][e_i, dsb] = sg

        if do_enew:
            pltpu.async_copy(ce_vs[slot], enew.at[pl.ds(cid * E + _off(j), B)],
                             s_en[slot])
        pltpu.async_copy(db_vs[slot], acc_sh.at[dsts_vs[slot]], s_scat, add=True)
        return 0

    idx_issue(0, 0)
    idx_wait(0)
    ce_issue(0, 0)
    adjust(0)
    gather_issue(0)
    idx_issue(1, 1)

    npair = nblk // 2

    def pair(g, carry):
        block(2 * g, 0, carry)
        return block(2 * g + 1, 1, carry)

    lax.fori_loop(0, npair, pair, 0)
    lax.fori_loop(2 * npair, nblk, lambda j, c_: block(j, 0, c_), 0)

    scat_wait()
    if do_enew:
        @pl.when(nblk % 2 == 1)
        def _():
            enew_wait(0)

        @pl.when(nblk % 2 == 0)
        def _():
            enew_wait(1)

    plsc.subcore_barrier()
    pltpu.sync_copy(
        acc_sh.at[pl.ds(base, DRAIN_ROWS)],
        accout.at[pl.ds(cid * N + base, DRAIN_ROWS)])

    @pl.when(sid == NUM_TILES - 1)
    def _():
        pltpu.sync_copy(
            acc_sh.at[pl.ds(N - ZROWS, ZROWS)],
            accout.at[pl.ds(cid * N + N - ZROWS, ZROWS)])


def _make_sc_edge_pass(do_enew):
    out_type = [_f32((2 * N, H))]
    if do_enew:
        out_type += [_f32((2 * E, HH))]
    scratch = [
        pltpu.VMEM((B,), jnp.int32),
        pltpu.VMEM((B,), jnp.int32),
        pltpu.VMEM((B,), jnp.int32),
        pltpu.VMEM((B,), jnp.int32),
        pltpu.VMEM((B,), jnp.int32),
        pltpu.VMEM((B,), jnp.int32),
        pltpu.VMEM((B, HH), F32),
        pltpu.VMEM((B, HH), F32),
        pltpu.VMEM((B, H), F32),
        pltpu.VMEM((B, H), F32),
        pltpu.VMEM((B, H), F32),
        pltpu.VMEM((B, H), F32),
        pltpu.SemaphoreType.DMA,
        pltpu.SemaphoreType.DMA,
        pltpu.SemaphoreType.DMA,
        pltpu.SemaphoreType.DMA,
        pltpu.SemaphoreType.DMA,
        pltpu.SemaphoreType.DMA,
        pltpu.SemaphoreType.DMA,
        pltpu.SemaphoreType.DMA,
        pltpu.SemaphoreType.DMA,
        pltpu.SemaphoreType.DMA,
        pltpu.SemaphoreType.DMA,
    ]
    if do_enew:
        scratch += [pltpu.SemaphoreType.DMA, pltpu.SemaphoreType.DMA]
    scratch += [pltpu.VMEM_SHARED((N, H), F32)]
    return pl.kernel(
        functools.partial(_sc_body, do_enew),
        out_type=out_type,
        mesh=plsc.VectorSubcoreMesh(core_axis_name="c", subcore_axis_name="s"),
        scratch_types=scratch,
    )


_sc_pass1 = _make_sc_edge_pass(True)
_sc_pass2 = _make_sc_edge_pass(False)


def kernel(h, e, state, params, edge_index):
    p = params
    l1, l2 = p["layers"]
    r2 = lambda b: b.reshape(1, H)
    src = edge_index[0]
    dst = edge_index[1]

    wf1, bf1, wf2, bf2 = _fold_weights(
        p["emb_e_W"], p["emb_e_b"].reshape(1, H),
        l1["C_W"], r2(l1["C_b"]), l2["C_W"], r2(l2["C_b"]))

    hh, ah, dbt, eht = _node_first(
        h, p["emb_h_W"], p["emb_h_b"].reshape(1, H),
        l1["A_W"], r2(l1["A_b"]), l1["B_W"], r2(l1["B_b"]),
        l1["D_W"], r2(l1["D_b"]), l1["E_W"], r2(l1["E_b"]))
    ce1 = _edge_linear1(e, wf1, bf1)
    acc1, enew1 = _sc_pass1(
        ce1.reshape(2 * E, HH), dbt.reshape(2 * N, H), eht, src, dst)
    raw1, ps1, ps21 = _node_update_a(ah, acc1.reshape(2, N, H))

    hh1, ah2, dbt2, eht2 = _node_mid(
        raw1, ps1, ps21, r2(l1["bn_h_g"]), r2(l1["bn_h_b"]), hh,
        l2["A_W"], r2(l2["A_b"]), l2["B_W"], r2(l2["B_b"]),
        l2["D_W"], r2(l2["D_b"]), l2["E_W"], r2(l2["E_b"]))
    eps1, eps2 = _edge_stats(enew1.reshape(2, E, HH))
    ce2 = _edge_linear2(e, enew1.reshape(2, E, HH), eps1, eps2, wf2, bf2,
                        l2["C_W"], r2(l1["bn_e_g"]), r2(l1["bn_e_b"]))
    (acc2,) = _sc_pass2(
        ce2.reshape(2 * E, HH), dbt2.reshape(2 * N, H), eht2, src, dst)
    raw2, ps2, ps22 = _node_update_a(ah2, acc2.reshape(2, N, H))
    hh2, psh = _node_update_b(raw2, ps2, ps22, r2(l2["bn_h_g"]),
                              r2(l2["bn_h_b"]), hh1)

    return _head(psh, state, p["l1_W"], p["l1_b"].reshape(1, 256),
                 p["l2_W"], p["l2_b"].reshape(1, 256),
                 p["l3_W"], p["l3_b"].reshape(1, 2))


# speedup vs baseline: 3.0483x; 1.0009x over previous
import functools

import jax
import jax.numpy as jnp
from jax import lax
from jax.experimental import pallas as pl
from jax.experimental.pallas import tpu as pltpu
from jax.experimental.pallas import tpu_sc as plsc

N = 10000
E = 320000
H = 128
HH = 64
NBLK = 2000
EBLK = 4000
B = 64
NUM_TILES = 16
DRAIN_ROWS = 624
ZROWS = 16
NUM_EBLK = E // B
F32 = jnp.float32


def _f32(x):
    return jax.ShapeDtypeStruct(x, F32)


def _fold_body(embW, embb, c1W, c1b, c2W, c2b, wf1, bf1, wf2, bf2):
    wf1[...] = jnp.dot(embW[...], c1W[...], preferred_element_type=F32)
    bf1[...] = jnp.dot(embb[...], c1W[...], preferred_element_type=F32) + c1b[...]
    wf2[...] = jnp.dot(embW[...], c2W[...], preferred_element_type=F32)
    bf2[...] = jnp.dot(embb[...], c2W[...], preferred_element_type=F32) + c2b[...]


def _fold_weights(embW, embb, c1W, c1b, c2W, c2b):
    return pl.pallas_call(
        _fold_body,
        out_shape=[_f32((16, H)), _f32((1, H)), _f32((16, H)), _f32((1, H))],
    )(embW, embb, c1W, c1b, c2W, c2b)


def _embed_body(h, w, b, out):
    out[...] = jnp.dot(h[...], w[...], preferred_element_type=F32) + b[...]


def _embed(h, w, b):
    return pl.pallas_call(
        _embed_body,
        grid=(N // NBLK,),
        in_specs=[
            pl.BlockSpec((NBLK, H), lambda i: (i, 0)),
            pl.BlockSpec((H, H), lambda i: (0, 0)),
            pl.BlockSpec((1, H), lambda i: (0, 0)),
        ],
        out_specs=pl.BlockSpec((NBLK, H), lambda i: (i, 0)),
        out_shape=_f32((N, H)),
    )(h, w, b)


def _mm_tables(x, aW, ab, bW, bb, dW, db_, eW, eb, ah, dbt, eht):
    ah[...] = jnp.dot(x, aW[...], preferred_element_type=F32) + ab[...]
    Bh = jnp.dot(x, bW[...], preferred_element_type=F32) + bb[...]
    Dh = jnp.dot(x, dW[...], preferred_element_type=F32) + db_[...]
    Eh = jnp.dot(x, eW[...], preferred_element_type=F32) + eb[...]
    dbt[0] = jnp.concatenate([Dh[:, :HH], Bh[:, :HH]], axis=1)
    dbt[1] = jnp.concatenate([Dh[:, HH:], Bh[:, HH:]], axis=1)
    eht[...] = Eh


def _node_first_body(h, embW, embb, aW, ab, bW, bb, dW, db_, eW, eb,
                     hh, ah, dbt, eht):
    x = jnp.dot(h[...], embW[...], preferred_element_type=F32) + embb[...]
    hh[...] = x
    _mm_tables(x, aW, ab, bW, bb, dW, db_, eW, eb, ah, dbt, eht)


def _node_first(h, embW, embb, aW, ab, bW, bb, dW, db_, eW, eb):
    return pl.pallas_call(
        _node_first_body,
        grid=(N // NBLK,),
        in_specs=[pl.BlockSpec((NBLK, H), lambda i: (i, 0))]
        + [pl.BlockSpec((H, H), lambda i: (0, 0)),
           pl.BlockSpec((1, H), lambda i: (0, 0))] * 5,
        out_specs=[
            pl.BlockSpec((NBLK, H), lambda i: (i, 0)),
            pl.BlockSpec((NBLK, H), lambda i: (i, 0)),
            pl.BlockSpec((2, NBLK, H), lambda i: (0, i, 0)),
            pl.BlockSpec((NBLK, H), lambda i: (i, 0)),
        ],
        out_shape=[_f32((N, H)), _f32((N, H)), _f32((2, N, H)), _f32((N, H))],
    )(h, embW, embb, aW, ab, bW, bb, dW, db_, eW, eb)


def _node_mid_body(raw, ps, ps2, g, b, hin, aW, ab, bW, bb, dW, db_, eW, eb,
                   hh, ah, dbt, eht):
    inv_n = 1.0 / float(N)
    m = jnp.sum(ps[...], axis=0) * inv_n
    v = jnp.sum(ps2[...], axis=0) * inv_n - m * m
    hn = (raw[...] - m) * lax.rsqrt(v + 1e-5) * g[...] + b[...]
    x = hin[...] + jnp.maximum(hn, 0.0)
    hh[...] = x
    _mm_tables(x, aW, ab, bW, bb, dW, db_, eW, eb, ah, dbt, eht)


def _node_mid(raw, ps, ps2, g, b, hin, aW, ab, bW, bb, dW, db_, eW, eb):
    return pl.pallas_call(
        _node_mid_body,
        grid=(N // NBLK,),
        in_specs=[
            pl.BlockSpec((NBLK, H), lambda i: (i, 0)),
            pl.BlockSpec((N // NBLK, 1, H), lambda i: (0, 0, 0)),
            pl.BlockSpec((N // NBLK, 1, H), lambda i: (0, 0, 0)),
            pl.BlockSpec((1, H), lambda i: (0, 0)),
            pl.BlockSpec((1, H), lambda i: (0, 0)),
            pl.BlockSpec((NBLK, H), lambda i: (i, 0)),
        ]
        + [pl.BlockSpec((H, H), lambda i: (0, 0)),
           pl.BlockSpec((1, H), lambda i: (0, 0))] * 4,
        out_specs=[
            pl.BlockSpec((NBLK, H), lambda i: (i, 0)),
            pl.BlockSpec((NBLK, H), lambda i: (i, 0)),
            pl.BlockSpec((2, NBLK, H), lambda i: (0, i, 0)),
            pl.BlockSpec((NBLK, H), lambda i: (i, 0)),
        ],
        out_shape=[_f32((N, H)), _f32((N, H)), _f32((2, N, H)), _f32((N, H))],
    )(raw, ps, ps2, g, b, hin, aW, ab, bW, bb, dW, db_, eW, eb)


def _edge_lin1_body(e, wf, bf, out):
    ce = jnp.dot(1.0 / e[...], wf[...], preferred_element_type=F32) + bf[...]
    out[0] = ce[:, :HH]
    out[1] = ce[:, HH:]


def _edge_linear1(e, wf, bf):
    return pl.pallas_call(
        _edge_lin1_body,
        grid=(E // EBLK,),
        in_specs=[
            pl.BlockSpec((EBLK, 16), lambda i: (i, 0)),
            pl.BlockSpec((16, H), lambda i: (0, 0)),
            pl.BlockSpec((1, H), lambda i: (0, 0)),
        ],
        out_specs=pl.BlockSpec((2, EBLK, HH), lambda i: (0, i, 0)),
        out_shape=_f32((2, E, HH)),
    )(e, wf, bf)


def _edge_stats_body(enew, ps, ps2):
    x0 = enew[0]
    x1 = enew[1]
    s1 = jnp.concatenate([jnp.sum(x0, axis=0, keepdims=True),
                          jnp.sum(x1, axis=0, keepdims=True)], axis=1)
    s2 = jnp.concatenate([jnp.sum(x0 * x0, axis=0, keepdims=True),
                          jnp.sum(x1 * x1, axis=0, keepdims=True)], axis=1)
    ps[...] = s1.reshape(1, 1, H)
    ps2[...] = s2.reshape(1, 1, H)


def _edge_stats(enew):
    return pl.pallas_call(
        _edge_stats_body,
        grid=(E // EBLK,),
        in_specs=[pl.BlockSpec((2, EBLK, HH), lambda i: (0, i, 0))],
        out_specs=[
            pl.BlockSpec((1, 1, H), lambda i: (i, 0, 0)),
            pl.BlockSpec((1, 1, H), lambda i: (i, 0, 0)),
        ],
        out_shape=[_f32((E // EBLK, 1, H)), _f32((E // EBLK, 1, H))],
    )(enew)


def _edge_lin2_body(e, enew, ps, ps2, wf, bf, cW, g, b, out):
    inv_e = 1.0 / float(E)
    s1 = jnp.sum(ps[...], axis=0)
    s2 = jnp.sum(ps2[...], axis=0)
    m0, m1 = s1[:, :HH] * inv_e, s1[:, HH:] * inv_e
    s2h0, s2h1 = s2[:, :HH], s2[:, HH:]
    v0 = s2h0 * inv_e - m0 * m0
    v1 = s2h1 * inv_e - m1 * m1
    r0 = (enew[0] - m0) * lax.rsqrt(v0 + 1e-5) * g[:, :HH] + b[:, :HH]
    r1 = (enew[1] - m1) * lax.rsqrt(v1 + 1e-5) * g[:, HH:] + b[:, HH:]
    r0 = jnp.maximum(r0, 0.0)
    r1 = jnp.maximum(r1, 0.0)
    ce = (jnp.dot(1.0 / e[...], wf[...], preferred_element_type=F32) + bf[...]
          + jnp.dot(r0, cW[:HH, :], preferred_element_type=F32)
          + jnp.dot(r1, cW[HH:, :], preferred_element_type=F32))
    out[0] = ce[:, :HH]
    out[1] = ce[:, HH:]


def _edge_linear2(e, enew, ps, ps2, wf, bf, cW, g, b):
    return pl.pallas_call(
        _edge_lin2_body,
        grid=(E // EBLK,),
        in_specs=[
            pl.BlockSpec((EBLK, 16), lambda i: (i, 0)),
            pl.BlockSpec((2, EBLK, HH), lambda i: (0, i, 0)),
            pl.BlockSpec((E // EBLK, 1, H), lambda i: (0, 0, 0)),
            pl.BlockSpec((E // EBLK, 1, H), lambda i: (0, 0, 0)),
            pl.BlockSpec((16, H), lambda i: (0, 0)),
            pl.BlockSpec((1, H), lambda i: (0, 0)),
            pl.BlockSpec((H, H), lambda i: (0, 0)),
            pl.BlockSpec((1, H), lambda i: (0, 0)),
            pl.BlockSpec((1, H), lambda i: (0, 0)),
        ],
        out_specs=pl.BlockSpec((2, EBLK, HH), lambda i: (0, i, 0)),
        out_shape=_f32((2, E, HH)),
    )(e, enew, ps, ps2, wf, bf, cW, g, b)


def _node_upd_a_body(ah, acc, raw, ps, ps2):
    num = jnp.concatenate([acc[0, :, :HH], acc[1, :, :HH]], axis=1)
    den = jnp.concatenate([acc[0, :, HH:], acc[1, :, HH:]], axis=1)
    r = ah[...] + num / (den + 1e-6)
    raw[...] = r
    ps[...] = jnp.sum(r, axis=0, keepdims=True).reshape(1, 1, H)
    ps2[...] = jnp.sum(r * r, axis=0, keepdims=True).reshape(1, 1, H)


def _node_update_a(ah, acc):
    return pl.pallas_call(
        _node_upd_a_body,
        grid=(N // NBLK,),
        in_specs=[
            pl.BlockSpec((NBLK, H), lambda i: (i, 0)),
            pl.BlockSpec((2, NBLK, H), lambda i: (0, i, 0)),
        ],
        out_specs=[
            pl.BlockSpec((NBLK, H), lambda i: (i, 0)),
            pl.BlockSpec((1, 1, H), lambda i: (i, 0, 0)),
            pl.BlockSpec((1, 1, H), lambda i: (i, 0, 0)),
        ],
        out_shape=[_f32((N, H)), _f32((N // NBLK, 1, H)), _f32((N // NBLK, 1, H))],
    )(ah, acc)


def _node_upd_b_body(raw, ps, ps2, g, b, hin, out, psh):
    inv_n = 1.0 / float(N)
    m = jnp.sum(ps[...], axis=0) * inv_n
    v = jnp.sum(ps2[...], axis=0) * inv_n - m * m
    hn = (raw[...] - m) * lax.rsqrt(v + 1e-5) * g[...] + b[...]
    r = hin[...] + jnp.maximum(hn, 0.0)
    out[...] = r
    psh[...] = jnp.sum(r, axis=0, keepdims=True).reshape(1, 1, H)


def _node_update_b(raw, ps, ps2, g, b, hin):
    return pl.pallas_call(
        _node_upd_b_body,
        grid=(N // NBLK,),
        in_specs=[
            pl.BlockSpec((NBLK, H), lambda i: (i, 0)),
            pl.BlockSpec((N // NBLK, 1, H), lambda i: (0, 0, 0)),
            pl.BlockSpec((N // NBLK, 1, H), lambda i: (0, 0, 0)),
            pl.BlockSpec((1, H), lambda i: (0, 0)),
            pl.BlockSpec((1, H), lambda i: (0, 0)),
            pl.BlockSpec((NBLK, H), lambda i: (i, 0)),
        ],
        out_specs=[
            pl.BlockSpec((NBLK, H), lambda i: (i, 0)),
            pl.BlockSpec((1, 1, H), lambda i: (i, 0, 0)),
        ],
        out_shape=[_f32((N, H)), _f32((N // NBLK, 1, H))],
    )(raw, ps, ps2, g, b, hin)


def _head_body(psh, st, w1, b1, w2, b2, w3, b3, out):
    hm = jnp.sum(psh[...], axis=0) * (1.0 / float(N))
    z = jnp.concatenate([hm, st[...]], axis=1)
    z = jnp.maximum(jnp.dot(z, w1[...], preferred_element_type=F32) + b1[...], 0.0)
    z = jnp.maximum(jnp.dot(z, w2[...], preferred_element_type=F32) + b2[...], 0.0)
    out[...] = jnp.tanh(jnp.dot(z, w3[...], preferred_element_type=F32) + b3[...])


def _head(psh, st, w1, b1, w2, b2, w3, b3):
    return pl.pallas_call(_head_body, out_shape=_f32((1, 2)))(
        psh, st, w1, b1, w2, b2, w3, b3)


def _sigmoid16(x):
    return 1.0 / (1.0 + jnp.exp(-x))


def _sc_body(do_enew, *refs):
    if do_enew:
        (ce, dbt, eht, src, dst, accout, enew,
         src_v0, src_v1, dst_v0, dst_v1, dsts_v0, dsts_v1,
         ce_v0, ce_v1, db_v0, db_v1, eh_v0, eh_v1,
         s_src0, s_src1, s_dst0, s_dst1, s_ce0, s_ce1,
         s_gdb0, s_gdb1, s_geh0, s_geh1, s_scat, s_en0, s_en1,
         acc_sh) = refs
        s_en = [s_en0, s_en1]
    else:
        (ce, dbt, eht, src, dst, accout,
         src_v0, src_v1, dst_v0, dst_v1, dsts_v0, dsts_v1,
         ce_v0, ce_v1, db_v0, db_v1, eh_v0, eh_v1,
         s_src0, s_src1, s_dst0, s_dst1, s_ce0, s_ce1,
         s_gdb0, s_gdb1, s_geh0, s_geh1, s_scat,
         acc_sh) = refs
    src_vs = [src_v0, src_v1]
    dst_vs = [dst_v0, dst_v1]
    dsts_vs = [dsts_v0, dsts_v1]
    ce_vs = [ce_v0, ce_v1]
    db_vs = [db_v0, db_v1]
    eh_vs = [eh_v0, eh_v1]
    s_src = [s_src0, s_src1]
    s_dst = [s_dst0, s_dst1]
    s_ce = [s_ce0, s_ce1]
    s_gdb = [s_gdb0, s_gdb1]
    s_geh = [s_geh0, s_geh1]

    cid = lax.axis_index("c")
    sid = lax.axis_index("s")

    def zfill(i, _):
        for c in range(H // 16):
            db_v0[i, pl.ds(c * 16, 16)] = jnp.zeros((16,), F32)
        return 0
    lax.fori_loop(0, ZROWS, zfill, 0)
    base = sid * DRAIN_ROWS
    zsrc = db_v0.at[pl.ds(0, ZROWS)]

    def zcopy(k, _):
        pltpu.sync_copy(zsrc, acc_sh.at[pl.ds(base + k * ZROWS, ZROWS)])
        return 0
    lax.fori_loop(0, DRAIN_ROWS // ZROWS, zcopy, 0)

    @pl.when(sid == NUM_TILES - 1)
    def _():
        pltpu.sync_copy(zsrc, acc_sh.at[pl.ds(N - ZROWS, ZROWS)])
    plsc.subcore_barrier()

    nblk = (NUM_EBLK - sid + NUM_TILES - 1) // NUM_TILES
    tbl_off = cid * N

    def _off(j):
        return (sid + j * NUM_TILES) * B

    def idx_issue(slot, j):
        o = _off(j)
        pltpu.async_copy(src.at[pl.ds(o, B)], src_vs[slot], s_src[slot])
        pltpu.async_copy(dst.at[pl.ds(o, B)], dst_vs[slot], s_dst[slot])

    def idx_wait(slot):
        pltpu.make_async_copy(src.at[pl.ds(0, B)], src_vs[slot], s_src[slot]).wait()
        pltpu.make_async_copy(dst.at[pl.ds(0, B)], dst_vs[slot], s_dst[slot]).wait()

    def ce_issue(slot, j):
        pltpu.async_copy(ce.at[pl.ds(cid * E + _off(j), B)], ce_vs[slot],
                         s_ce[slot])

    def ce_wait(slot):
        pltpu.make_async_copy(ce.at[pl.ds(0, B)], ce_vs[slot], s_ce[slot]).wait()

    def adjust(slot):
        for c in range(B // 16):
            cs = pl.ds(c * 16, 16)
            src_vs[slot][cs] = src_vs[slot][cs] + tbl_off
            dsts_vs[slot][cs] = dst_vs[slot][cs]

    def gather_issue(slot):
        pltpu.async_copy(dbt.at[src_vs[slot]], db_vs[slot], s_gdb[slot])
        pltpu.async_copy(eht.at[dsts_vs[slot]], eh_vs[slot], s_geh[slot])

    def gather_wait(slot):
        pltpu.make_async_copy(dbt.at[src_vs[slot]], db_vs[slot],
                              s_gdb[slot]).wait()
        pltpu.make_async_copy(eht.at[dsts_vs[slot]], eh_vs[slot],
                              s_geh[slot]).wait()

    def scat_wait():
        pltpu.make_async_copy(db_v0, acc_sh.at[dsts_v0], s_scat).wait()

    def enew_wait(slot):
        pltpu.make_async_copy(ce_vs[slot], enew.at[pl.ds(0, B)], s_en[slot]).wait()

    def block(j, slot, _carry):
        oth = 1 - slot

        @pl.when(j + 1 < nblk)
        def _():
            idx_wait(oth)

        @pl.when(j >= 1)
        def _():
            scat_wait()
        if do_enew:
            @pl.when(j >= 1)
            def _():
                enew_wait(oth)

        @pl.when(j + 1 < nblk)
        def _():
            ce_issue(oth, j + 1)
            adjust(oth)
            gather_issue(oth)

        gather_wait(slot)
        ce_wait(slot)

        @pl.when(j + 2 < nblk)
        def _():
            idx_issue(slot, j + 2)

        @plsc.parallel_loop(0, B, 1, unroll=4)
        def edge(e_i):
            for c in range(HH // 16):
                ds = pl.ds(c * 16, 16)
                dsb = pl.ds(HH + c * 16, 16)
                en = (ce_vs[slot][e_i, ds] + db_vs[slot][e_i, ds]
                      + eh_vs[slot][e_i, pl.ds(cid * HH + c * 16, 16)])
                if do_enew:
                    ce_vs[slot][e_i, ds] = en
                sg = _sigmoid16(en)
                num = sg * db_vs[slot][e_i, dsb]
                db_vs[slot][e_i, ds] = num
                db_vs[slot][e_i, dsb] = sg

        if do_enew:
            pltpu.async_copy(ce_vs[slot], enew.at[pl.ds(cid * E + _off(j), B)],
                             s_en[slot])
        pltpu.async_copy(db_vs[slot], acc_sh.at[dsts_vs[slot]], s_scat, add=True)
        return 0

    idx_issue(0, 0)
    idx_wait(0)
    ce_issue(0, 0)
    adjust(0)
    gather_issue(0)
    idx_issue(1, 1)

    npair = nblk // 2

    def pair(g, carry):
        block(2 * g, 0, carry)
        return block(2 * g + 1, 1, carry)

    lax.fori_loop(0, npair, pair, 0)
    lax.fori_loop(2 * npair, nblk, lambda j, c_: block(j, 0, c_), 0)

    scat_wait()
    if do_enew:
        @pl.when(nblk % 2 == 1)
        def _():
            enew_wait(0)

        @pl.when(nblk % 2 == 0)
        def _():
            enew_wait(1)

    plsc.subcore_barrier()
    pltpu.sync_copy(
        acc_sh.at[pl.ds(base, DRAIN_ROWS)],
        accout.at[pl.ds(cid * N + base, DRAIN_ROWS)])

    @pl.when(sid == NUM_TILES - 1)
    def _():
        pltpu.sync_copy(
            acc_sh.at[pl.ds(N - ZROWS, ZROWS)],
            accout.at[pl.ds(cid * N + N - ZROWS, ZROWS)])


def _make_sc_edge_pass(do_enew):
    out_type = [_f32((2 * N, H))]
    if do_enew:
        out_type += [_f32((2 * E, HH))]
    scratch = [
        pltpu.VMEM((B,), jnp.int32),
        pltpu.VMEM((B,), jnp.int32),
        pltpu.VMEM((B,), jnp.int32),
        pltpu.VMEM((B,), jnp.int32),
        pltpu.VMEM((B,), jnp.int32),
        pltpu.VMEM((B,), jnp.int32),
        pltpu.VMEM((B, HH), F32),
        pltpu.VMEM((B, HH), F32),
        pltpu.VMEM((B, H), F32),
        pltpu.VMEM((B, H), F32),
        pltpu.VMEM((B, H), F32),
        pltpu.VMEM((B, H), F32),
        pltpu.SemaphoreType.DMA,
        pltpu.SemaphoreType.DMA,
        pltpu.SemaphoreType.DMA,
        pltpu.SemaphoreType.DMA,
        pltpu.SemaphoreType.DMA,
        pltpu.SemaphoreType.DMA,
        pltpu.SemaphoreType.DMA,
        pltpu.SemaphoreType.DMA,
        pltpu.SemaphoreType.DMA,
        pltpu.SemaphoreType.DMA,
        pltpu.SemaphoreType.DMA,
    ]
    if do_enew:
        scratch += [pltpu.SemaphoreType.DMA, pltpu.SemaphoreType.DMA]
    scratch += [pltpu.VMEM_SHARED((N, H), F32)]
    return pl.kernel(
        functools.partial(_sc_body, do_enew),
        out_type=out_type,
        mesh=plsc.VectorSubcoreMesh(core_axis_name="c", subcore_axis_name="s"),
        scratch_types=scratch,
    )


_sc_pass1 = _make_sc_edge_pass(True)
_sc_pass2 = _make_sc_edge_pass(False)


def kernel(h, e, state, params, edge_index):
    p = params
    l1, l2 = p["layers"]
    r2 = lambda b: b.reshape(1, H)
    src = edge_index[0]
    dst = edge_index[1]

    wf1, bf1, wf2, bf2 = _fold_weights(
        p["emb_e_W"], p["emb_e_b"].reshape(1, H),
        l1["C_W"], r2(l1["C_b"]), l2["C_W"], r2(l2["C_b"]))

    hh, ah, dbt, eht = _node_first(
        h, p["emb_h_W"], p["emb_h_b"].reshape(1, H),
        l1["A_W"], r2(l1["A_b"]), l1["B_W"], r2(l1["B_b"]),
        l1["D_W"], r2(l1["D_b"]), l1["E_W"], r2(l1["E_b"]))
    ce1 = _edge_linear1(e, wf1, bf1)
    acc1, enew1 = _sc_pass1(
        ce1.reshape(2 * E, HH), dbt.reshape(2 * N, H), eht, src, dst)
    raw1, ps1, ps21 = _node_update_a(ah, acc1.reshape(2, N, H))

    hh1, ah2, dbt2, eht2 = _node_mid(
        raw1, ps1, ps21, r2(l1["bn_h_g"]), r2(l1["bn_h_b"]), hh,
        l2["A_W"], r2(l2["A_b"]), l2["B_W"], r2(l2["B_b"]),
        l2["D_W"], r2(l2["D_b"]), l2["E_W"], r2(l2["E_b"]))
    eps1, eps2 = _edge_stats(enew1.reshape(2, E, HH))
    ce2 = _edge_linear2(e, enew1.reshape(2, E, HH), eps1, eps2, wf2, bf2,
                        l2["C_W"], r2(l1["bn_e_g"]), r2(l1["bn_e_b"]))
    (acc2,) = _sc_pass2(
        ce2.reshape(2 * E, HH), dbt2.reshape(2 * N, H), eht2, src, dst)
    raw2, ps2, ps22 = _node_update_a(ah2, acc2.reshape(2, N, H))
    hh2, psh = _node_update_b(raw2, ps2, ps22, r2(l2["bn_h_g"]),
                              r2(l2["bn_h_b"]), hh1)

    return _head(psh, state, p["l1_W"], p["l1_b"].reshape(1, 256),
                 p["l2_W"], p["l2_b"].reshape(1, 256),
                 p["l3_W"], p["l3_b"].reshape(1, 2))
